# Initial kernel scaffold; baseline (speedup 1.0000x reference)
import jax
import jax.numpy as jnp
from jax.experimental import pallas as pl


def kernel(x, edge_index, edge_type, rgcn_w, rgcn_root, rgcn_bias, wq, bq, wk, bk, wv, bv, wskip, bskip, fc_w, fc_b):
    raise NotImplementedError("write your pallas kernel here")


# speedup vs baseline: 13.8109x; 13.8109x over previous
import functools

import jax
import jax.numpy as jnp
from jax import lax
from jax.experimental import pallas as pl
from jax.experimental.pallas import tpu as pltpu
from jax.experimental.pallas import tpu_sc as plsc

N = 10000
E = 160000
RELS = 6
DH = 128
NPAD = 10240
EPAD = 163840
CH = 128
CNTP = 60416
NS = 16
NB = 1000
CHA = 64


def _mm7_body(x_ref, w_ref, o_ref):
    o_ref[0] = jnp.dot(x_ref[...], w_ref[0], preferred_element_type=jnp.float32)


def _qkv_body(agg_ref, xroot_ref, bias_ref, wcat_ref, bcat_ref,
              q01_ref, q23_ref, k01_ref, k23_ref, v01_ref, v23_ref, skip_ref):
    h = agg_ref[0] + agg_ref[1] + xroot_ref[...] + bias_ref[...]
    h = jnp.maximum(h, 0.0)
    y = jnp.dot(h, wcat_ref[...], preferred_element_type=jnp.float32) + bcat_ref[...]
    q01_ref[...] = y[:, 0:128]
    q23_ref[...] = y[:, 128:256]
    k01_ref[...] = y[:, 256:384]
    k23_ref[...] = y[:, 384:512]
    v01_ref[...] = y[:, 512:640]
    v23_ref[...] = y[:, 640:768]
    skip_ref[...] = y[:, 768:1024]


def _final_body(num_ref, asum_ref, skip_ref, bskip_ref, fcw_ref, fcb_ref,
                o_ref, acc_ref):
    i = pl.program_id(0)
    blk = 1024
    a = asum_ref[...]
    a00 = jnp.maximum(a[0, 0, :], 1e-30).reshape(blk, 1)
    a01 = jnp.maximum(a[0, 1, :], 1e-30).reshape(blk, 1)
    a10 = jnp.maximum(a[1, 0, :], 1e-30).reshape(blk, 1)
    a11 = jnp.maximum(a[1, 1, :], 1e-30).reshape(blk, 1)
    nm = num_ref[...]
    o = jnp.concatenate(
        [nm[0, :, 0:64] / a00, nm[0, :, 64:128] / a01,
         nm[1, :, 0:64] / a10, nm[1, :, 64:128] / a11], axis=1)
    o = jnp.maximum(o + skip_ref[...] + bskip_ref[...], 0.0)
    rows = i * blk + lax.broadcasted_iota(jnp.int32, (blk, 1), 0)
    o = jnp.where(rows < N, o, 0.0)
    part = jnp.sum(o, axis=0, keepdims=True)

    @pl.when(i == 0)
    def _():
        acc_ref[...] = part

    @pl.when(i > 0)
    def _():
        acc_ref[...] += part

    @pl.when(i == (NPAD // blk) - 1)
    def _():
        o_ref[...] = jnp.dot(acc_ref[...] * (1.0 / N), fcw_ref[...],
                             preferred_element_type=jnp.float32) + fcb_ref[...]


_SC_MESH = plsc.VectorSubcoreMesh(core_axis_name="c", subcore_axis_name="s")
_Z16 = functools.partial(jnp.zeros, (16,), jnp.float32)


@functools.partial(
    pl.kernel,
    out_type=jax.ShapeDtypeStruct((2, NPAD, DH), jnp.float32),
    mesh=_SC_MESH,
    scratch_types=[
        pltpu.VMEM((CH,), jnp.int32),
        pltpu.VMEM((CH,), jnp.int32),
        pltpu.VMEM((CH,), jnp.int32),
        pltpu.VMEM((CH,), jnp.int32),
        pltpu.VMEM((CH,), jnp.int32),
        pltpu.VMEM((CH,), jnp.float32),
        pltpu.VMEM((CH, DH), jnp.float32),
        pltpu.VMEM((CH,), jnp.float32),
        pltpu.VMEM((CNTP // NS,), jnp.float32),
        pltpu.VMEM_SHARED((NPAD, DH), jnp.float32),
        pltpu.VMEM_SHARED((CNTP,), jnp.float32),
        pltpu.SemaphoreType.DMA,
    ],
)
def _rgcn_sc(src_hbm, dsts_hbm, et_hbm, xw_hbm, out_hbm,
             srcb, dstb, etb, gidxb, segb, onesb, rows, invw, zb1,
             agg_sp, cnt_sp, sem):
    c = lax.axis_index("c")
    s = lax.axis_index("s")

    def zrow(e, _):
        for k in range(8):
            rows[e, pl.ds(k * 16, 16)] = _Z16()
        return 0
    lax.fori_loop(0, CH, zrow, 0)

    def zzb(i, _):
        zb1[pl.ds(i * 16, 16)] = _Z16()
        return 0
    lax.fori_loop(0, (CNTP // NS) // 16, zzb, 0)

    for j in range(NPAD // NS // CH):
        pltpu.sync_copy(rows, agg_sp.at[pl.ds(s * (NPAD // NS) + j * CH, CH)])
    pltpu.sync_copy(zb1, cnt_sp.at[pl.ds(s * (CNTP // NS), CNTP // NS)])
    for k in range(8):
        onesb[pl.ds(k * 16, 16)] = _Z16() + 1.0
    plsc.subcore_barrier()

    def cnt_chunk(j, _):
        b = s * (EPAD // NS) + j * CH
        pltpu.sync_copy(dsts_hbm.at[pl.ds(b, CH)], dstb)
        pltpu.sync_copy(et_hbm.at[pl.ds(b, CH)], etb)
        for k in range(8):
            sl = pl.ds(k * 16, 16)
            segb[sl] = dstb[sl] * RELS + etb[sl]
        pltpu.sync_copy(onesb, cnt_sp.at[segb], add=True)
        return 0
    lax.fori_loop(0, EPAD // NS // CH, cnt_chunk, 0)
    plsc.subcore_barrier()

    pltpu.sync_copy(cnt_sp.at[pl.ds(s * (CNTP // NS), CNTP // NS)], zb1)

    def inv_loop(i, _):
        sl = pl.ds(i * 16, 16)
        zb1[sl] = 1.0 / jnp.maximum(zb1[sl], 1.0)
        return 0
    lax.fori_loop(0, (CNTP // NS) // 16, inv_loop, 0)
    pltpu.sync_copy(zb1, cnt_sp.at[pl.ds(s * (CNTP // NS), CNTP // NS)])
    plsc.subcore_barrier()

    def agg_chunk(j, _):
        b = (c * NS + s) * (EPAD // 32) + j * CH
        pltpu.sync_copy(src_hbm.at[pl.ds(b, CH)], srcb)
        pltpu.sync_copy(dsts_hbm.at[pl.ds(b, CH)], dstb)
        pltpu.sync_copy(et_hbm.at[pl.ds(b, CH)], etb)
        for k in range(8):
            sl = pl.ds(k * 16, 16)
            gidxb[sl] = etb[sl] * N + srcb[sl]
            segb[sl] = dstb[sl] * RELS + etb[sl]
        pltpu.async_copy(xw_hbm.at[gidxb], rows, sem).wait()
        pltpu.async_copy(cnt_sp.at[segb], invw, sem).wait()

        def escale(g, _):
            iw16 = invw[pl.ds(g * 16, 16)]
            for e2 in range(16):
                w_e = iw16[e2]
                row = g * 16 + e2
                for k in range(8):
                    sl = pl.ds(k * 16, 16)
                    rows[row, sl] = rows[row, sl] * w_e
            return 0
        lax.fori_loop(0, CH // 16, escale, 0)
        pltpu.sync_copy(rows, agg_sp.at[dstb], add=True)
        return 0
    lax.fori_loop(0, EPAD // 32 // CH, agg_chunk, 0)
    plsc.subcore_barrier()

    for j in range(NPAD // NS // CH):
        r0 = s * (NPAD // NS) + j * CH
        pltpu.sync_copy(agg_sp.at[pl.ds(r0, CH)], rows)
        pltpu.sync_copy(rows, out_hbm.at[c, pl.ds(r0, CH)])


@functools.partial(
    pl.kernel,
    out_type=(jax.ShapeDtypeStruct((2, NPAD, DH), jnp.float32),
              jax.ShapeDtypeStruct((2, 2, NPAD), jnp.float32)),
    mesh=_SC_MESH,
    scratch_types=[
        pltpu.VMEM((CHA,), jnp.int32),
        pltpu.VMEM((CHA,), jnp.int32),
        pltpu.VMEM((CHA,), jnp.int32),
        pltpu.VMEM((CHA, DH), jnp.float32),
        pltpu.VMEM((CHA, DH), jnp.float32),
        pltpu.VMEM((EPAD // NS,), jnp.float32),
        pltpu.VMEM((EPAD // NS,), jnp.float32),
        pltpu.VMEM((CHA,), jnp.float32),
        pltpu.VMEM((CHA,), jnp.float32),
        pltpu.VMEM((16,), jnp.float32),
        pltpu.VMEM((NS, 16), jnp.float32),
        pltpu.VMEM((NPAD // NS,), jnp.float32),
        pltpu.VMEM((16, 32), jnp.float32),
        pltpu.VMEM((16, 32), jnp.float32),
        pltpu.VMEM_SHARED((NPAD, DH), jnp.float32),
        pltpu.VMEM_SHARED((NPAD,), jnp.float32),
        pltpu.VMEM_SHARED((NPAD,), jnp.float32),
        pltpu.VMEM_SHARED((NS, 16), jnp.float32),
        pltpu.SemaphoreType.DMA,
    ],
)
def _attn_sc(src_hbm, dstg_hbm, dsts_hbm,
             q01, q23, k01, k23, v01, v23,
             num_out, asum_out,
             srcb, dstgb, dstsb, qrows, krows, al0, al1, aexp0, aexp1,
             mbuf, mall, zb, pad0, pad1, num_sp, as0_sp, as1_sp, mx_sp, sem):

    def lanesum(v, pad, slot):
        for sh in (8, 4, 2, 1):
            pad[slot, pl.ds(0, 16)] = v
            v = v + pad[slot, pl.ds(sh, 16)]
        return v[0]

    c = lax.axis_index("c")
    s = lax.axis_index("s")
    tbase = s * (EPAD // NS)
    nchunks = EPAD // NS // CHA

    def zrow(e, _):
        for k in range(8):
            qrows[e, pl.ds(k * 16, 16)] = _Z16()
        return 0
    lax.fori_loop(0, CHA, zrow, 0)

    def zzb(i, _):
        zb[pl.ds(i * 16, 16)] = _Z16()
        return 0
    lax.fori_loop(0, (NPAD // NS) // 16, zzb, 0)
    for r in range(16):
        pad0[r, pl.ds(0, 16)] = _Z16()
        pad0[r, pl.ds(16, 16)] = _Z16()
        pad1[r, pl.ds(0, 16)] = _Z16()
        pad1[r, pl.ds(16, 16)] = _Z16()

    for j in range(NPAD // NS // CHA):
        pltpu.sync_copy(qrows, num_sp.at[pl.ds(s * (NPAD // NS) + j * CHA, CHA)])
    pltpu.sync_copy(zb, as0_sp.at[pl.ds(s * (NPAD // NS), NPAD // NS)])
    pltpu.sync_copy(zb, as1_sp.at[pl.ds(s * (NPAD // NS), NPAD // NS)])
    plsc.subcore_barrier()

    def phase_a(qt, kt):
        def chunk(j, m):
            b = tbase + j * CHA
            pltpu.sync_copy(src_hbm.at[pl.ds(b, CHA)], srcb)
            pltpu.sync_copy(dstg_hbm.at[pl.ds(b, CHA)], dstgb)
            pltpu.async_copy(qt.at[dstgb], qrows, sem).wait()
            pltpu.async_copy(kt.at[srcb], krows, sem).wait()
            lane = lax.iota(jnp.int32, 16)

            def egrp(g, m):
                z0 = _Z16()
                z1 = _Z16()
                for e2 in range(16):
                    row = g * 16 + e2
                    p0 = qrows[row, pl.ds(0, 16)] * krows[row, pl.ds(0, 16)]
                    for k in range(1, 4):
                        sl = pl.ds(k * 16, 16)
                        p0 = p0 + qrows[row, sl] * krows[row, sl]
                    p1 = qrows[row, pl.ds(64, 16)] * krows[row, pl.ds(64, 16)]
                    for k in range(5, 8):
                        sl = pl.ds(k * 16, 16)
                        p1 = p1 + qrows[row, sl] * krows[row, sl]
                    s0 = lanesum(p0, pad0, e2)
                    s1 = lanesum(p1, pad1, e2)
                    z0 = jnp.where(lane == e2, s0, z0)
                    z1 = jnp.where(lane == e2, s1, z1)
                al0[pl.ds(j * CHA + g * 16, 16)] = z0
                al1[pl.ds(j * CHA + g * 16, 16)] = z1
                return jnp.maximum(m, jnp.maximum(z0, z1))
            return lax.fori_loop(0, CHA // 16, egrp, m)
        m = lax.fori_loop(0, nchunks, chunk,
                          jnp.full((16,), -3.0e38, jnp.float32))
        mbuf[pl.ds(0, 16)] = m

    @pl.when(c == 0)
    def _():
        phase_a(q01, k01)

    @pl.when(c == 1)
    def _():
        phase_a(q23, k23)

    pltpu.sync_copy(mbuf, mx_sp.at[s])
    plsc.subcore_barrier()
    pltpu.sync_copy(mx_sp, mall)
    gv = mall[0, pl.ds(0, 16)]
    for t in range(1, NS):
        gv = jnp.maximum(gv, mall[t, pl.ds(0, 16)])
    gmax = gv[0]
    for t in range(1, 16):
        gmax = jnp.maximum(gmax, gv[t])

    def phase_b(vt):
        def chunk(j, _):
            b = tbase + j * CHA
            pltpu.sync_copy(src_hbm.at[pl.ds(b, CHA)], srcb)
            pltpu.sync_copy(dsts_hbm.at[pl.ds(b, CHA)], dstsb)
            pltpu.async_copy(vt.at[srcb], qrows, sem).wait()
            for k in range(CHA // 16):
                sl = pl.ds(k * 16, 16)
                asl = pl.ds(j * CHA + k * 16, 16)
                aexp0[sl] = jnp.exp(al0[asl] - gmax)
                aexp1[sl] = jnp.exp(al1[asl] - gmax)

            def escale(g, _):
                a0v = aexp0[pl.ds(g * 16, 16)]
                a1v = aexp1[pl.ds(g * 16, 16)]
                for e2 in range(16):
                    row = g * 16 + e2
                    a0 = a0v[e2]
                    a1 = a1v[e2]
                    for k in range(4):
                        sl = pl.ds(k * 16, 16)
                        qrows[row, sl] = qrows[row, sl] * a0
                    for k in range(4, 8):
                        sl = pl.ds(k * 16, 16)
                        qrows[row, sl] = qrows[row, sl] * a1
                return 0
            lax.fori_loop(0, CHA // 16, escale, 0)
            pltpu.sync_copy(aexp0, as0_sp.at[dstsb], add=True)
            pltpu.sync_copy(aexp1, as1_sp.at[dstsb], add=True)
            pltpu.sync_copy(qrows, num_sp.at[dstsb], add=True)
            return 0
        lax.fori_loop(0, nchunks, chunk, 0)

    @pl.when(c == 0)
    def _():
        phase_b(v01)

    @pl.when(c == 1)
    def _():
        phase_b(v23)

    plsc.subcore_barrier()

    for j in range(NPAD // NS // CHA):
        r0 = s * (NPAD // NS) + j * CHA
        pltpu.sync_copy(num_sp.at[pl.ds(r0, CHA)], krows)
        pltpu.sync_copy(krows, num_out.at[c, pl.ds(r0, CHA)])
    a0 = s * (NPAD // NS)
    pltpu.sync_copy(as0_sp.at[pl.ds(a0, NPAD // NS)], zb)
    pltpu.sync_copy(zb, asum_out.at[c, 0, pl.ds(a0, NPAD // NS)])
    pltpu.sync_copy(as1_sp.at[pl.ds(a0, NPAD // NS)], zb)
    pltpu.sync_copy(zb, asum_out.at[c, 1, pl.ds(a0, NPAD // NS)])


def kernel(x, edge_index, edge_type, rgcn_w, rgcn_root, rgcn_bias,
           wq, bq, wk, bk, wv, bv, wskip, bskip, fc_w, fc_b):
    src = edge_index[0].astype(jnp.int32)
    dst = edge_index[1].astype(jnp.int32)
    et = edge_type.astype(jnp.int32)

    pcnt = EPAD - E
    pidx = jnp.arange(pcnt, dtype=jnp.int32)
    src_g = jnp.concatenate([src, (pidx * 997) % N])
    dst_g = jnp.concatenate([dst, (pidx * 1013 + 7) % N])
    dst_s = jnp.concatenate([dst, N + (pidx % 16)])
    et_g = jnp.concatenate([et, jnp.zeros((pcnt,), jnp.int32)])

    w7 = jnp.concatenate([rgcn_w, rgcn_root[None]], axis=0)
    xw7 = pl.pallas_call(
        _mm7_body,
        grid=(N // NB, 7),
        in_specs=[pl.BlockSpec((NB, 768), lambda i, r: (i, 0)),
                  pl.BlockSpec((1, 768, DH), lambda i, r: (r, 0, 0))],
        out_specs=pl.BlockSpec((1, NB, DH), lambda i, r: (r, i, 0)),
        out_shape=jax.ShapeDtypeStruct((7, N, DH), jnp.float32),
    )(x, w7)
    xw_flat = xw7[:RELS].reshape(RELS * N, DH)
    xroot = xw7[RELS]

    agg2 = _rgcn_sc(src_g, dst_s, et_g, xw_flat)

    wcat = jnp.concatenate([wq * 0.125, wk, wv, wskip], axis=1)
    bcat = jnp.concatenate([bq * 0.125, bk, bv, bskip])[None]
    outs = pl.pallas_call(
        _qkv_body,
        grid=(N // NB,),
        in_specs=[pl.BlockSpec((2, NB, DH), lambda i: (0, i, 0)),
                  pl.BlockSpec((NB, DH), lambda i: (i, 0)),
                  pl.BlockSpec((1, DH), lambda i: (0, 0)),
                  pl.BlockSpec((DH, 1024), lambda i: (0, 0)),
                  pl.BlockSpec((1, 1024), lambda i: (0, 0))],
        out_specs=[pl.BlockSpec((NB, DH), lambda i: (i, 0))] * 6
        + [pl.BlockSpec((NB, 256), lambda i: (i, 0))],
        out_shape=[jax.ShapeDtypeStruct((N, DH), jnp.float32)] * 6
        + [jax.ShapeDtypeStruct((N, 256), jnp.float32)],
    )(agg2, xroot, rgcn_bias[None], wcat, bcat)
    q01, q23, k01, k23, v01, v23, skip = outs

    num2, asum2 = _attn_sc(src_g, dst_g, dst_s, q01, q23, k01, k23,
                           v01, v23)

    skip_p = jnp.pad(skip, ((0, NPAD - N), (0, 0)))
    fcw_p = jnp.pad(fc_w, ((0, 0), (0, 125)))
    fcb_p = jnp.pad(fc_b, (0, 125))[None]
    out = pl.pallas_call(
        _final_body,
        grid=(NPAD // 1024,),
        in_specs=[pl.BlockSpec((2, 1024, DH), lambda i: (0, i, 0)),
                  pl.BlockSpec((2, 2, 1024), lambda i: (0, 0, i)),
                  pl.BlockSpec((1024, 256), lambda i: (i, 0)),
                  pl.BlockSpec((1, 256), lambda i: (0, 0)),
                  pl.BlockSpec((256, 128), lambda i: (0, 0)),
                  pl.BlockSpec((1, 128), lambda i: (0, 0))],
        out_specs=pl.BlockSpec((1, 128), lambda i: (0, 0)),
        out_shape=jax.ShapeDtypeStruct((1, 128), jnp.float32),
        scratch_shapes=[pltpu.VMEM((1, 256), jnp.float32)],
    )(num2, asum2, skip_p, bskip[None], fcw_p, fcb_p)
    return out[:, :3]


# speedup vs baseline: 21.6877x; 1.5703x over previous
import functools

import jax
import jax.numpy as jnp
from jax import lax
from jax.experimental import pallas as pl
from jax.experimental.pallas import tpu as pltpu
from jax.experimental.pallas import tpu_sc as plsc

N = 10000
E = 160000
RELS = 6
DH = 128
NPAD = 10240
EPAD = 163840
CH = 128
CNTP = 60416
NS = 16
NB = 1000
CHA = 64
GRP = 1024


def _mm7_body(x_ref, w_ref, o_ref):
    o_ref[0] = jnp.dot(x_ref[...], w_ref[0], preferred_element_type=jnp.float32)


def _qkv_body(agg_ref, xroot_ref, bias_ref, wcat_ref, bcat_ref,
              q01_ref, q23_ref, k01_ref, k23_ref, v01_ref, v23_ref, skip_ref):
    h = agg_ref[0] + agg_ref[1] + xroot_ref[...] + bias_ref[...]
    h = jnp.maximum(h, 0.0)
    y = jnp.dot(h, wcat_ref[...], preferred_element_type=jnp.float32) + bcat_ref[...]
    q01_ref[...] = y[:, 0:128]
    q23_ref[...] = y[:, 128:256]
    k01_ref[...] = y[:, 256:384]
    k23_ref[...] = y[:, 384:512]
    v01_ref[...] = y[:, 512:640]
    v23_ref[...] = y[:, 640:768]
    skip_ref[...] = y[:, 768:1024]


def _final_body(num_ref, asum_ref, skip_ref, bskip_ref, fcw_ref, fcb_ref,
                o_ref, acc_ref):
    i = pl.program_id(0)
    blk = 1024
    a = asum_ref[...]
    a00 = jnp.maximum(a[0, 0, :], 1e-30).reshape(blk, 1)
    a01 = jnp.maximum(a[0, 1, :], 1e-30).reshape(blk, 1)
    a10 = jnp.maximum(a[1, 0, :], 1e-30).reshape(blk, 1)
    a11 = jnp.maximum(a[1, 1, :], 1e-30).reshape(blk, 1)
    nm = num_ref[...]
    o = jnp.concatenate(
        [nm[0, :, 0:64] / a00, nm[0, :, 64:128] / a01,
         nm[1, :, 0:64] / a10, nm[1, :, 64:128] / a11], axis=1)
    o = jnp.maximum(o + skip_ref[...] + bskip_ref[...], 0.0)
    rows = i * blk + lax.broadcasted_iota(jnp.int32, (blk, 1), 0)
    o = jnp.where(rows < N, o, 0.0)
    part = jnp.sum(o, axis=0, keepdims=True)

    @pl.when(i == 0)
    def _():
        acc_ref[...] = part

    @pl.when(i > 0)
    def _():
        acc_ref[...] += part

    @pl.when(i == (NPAD // blk) - 1)
    def _():
        o_ref[...] = jnp.dot(acc_ref[...] * (1.0 / N), fcw_ref[...],
                             preferred_element_type=jnp.float32) + fcb_ref[...]


_SC_MESH = plsc.VectorSubcoreMesh(core_axis_name="c", subcore_axis_name="s")
_Z16 = functools.partial(jnp.zeros, (16,), jnp.float32)


@functools.partial(
    pl.kernel,
    out_type=jax.ShapeDtypeStruct((2, NPAD, DH), jnp.float32),
    mesh=_SC_MESH,
    scratch_types=[
        pltpu.VMEM((CH,), jnp.int32),
        pltpu.VMEM((CH,), jnp.int32),
        pltpu.VMEM((CH,), jnp.int32),
        pltpu.VMEM((CH,), jnp.int32),
        pltpu.VMEM((CH,), jnp.int32),
        pltpu.VMEM((CH,), jnp.float32),
        pltpu.VMEM((CH, DH), jnp.float32),
        pltpu.VMEM((CH,), jnp.float32),
        pltpu.VMEM((CNTP // NS,), jnp.float32),
        pltpu.VMEM_SHARED((NPAD, DH), jnp.float32),
        pltpu.VMEM_SHARED((CNTP,), jnp.float32),
        pltpu.SemaphoreType.DMA,
    ],
)
def _rgcn_sc(src_hbm, dsts_hbm, et_hbm, xw_hbm, out_hbm,
             srcb, dstb, etb, gidxb, segb, onesb, rows, invw, zb1,
             agg_sp, cnt_sp, sem):
    c = lax.axis_index("c")
    s = lax.axis_index("s")

    def zrow(e, _):
        for k in range(8):
            rows[e, pl.ds(k * 16, 16)] = _Z16()
        return 0
    lax.fori_loop(0, CH, zrow, 0)

    def zzb(i, _):
        zb1[pl.ds(i * 16, 16)] = _Z16()
        return 0
    lax.fori_loop(0, (CNTP // NS) // 16, zzb, 0)

    for j in range(NPAD // NS // CH):
        pltpu.sync_copy(rows, agg_sp.at[pl.ds(s * (NPAD // NS) + j * CH, CH)])
    pltpu.sync_copy(zb1, cnt_sp.at[pl.ds(s * (CNTP // NS), CNTP // NS)])
    for k in range(8):
        onesb[pl.ds(k * 16, 16)] = _Z16() + 1.0
    plsc.subcore_barrier()

    def cnt_chunk(j, _):
        b = s * (EPAD // NS) + j * CH
        pltpu.sync_copy(dsts_hbm.at[pl.ds(b, CH)], dstb)
        pltpu.sync_copy(et_hbm.at[pl.ds(b, CH)], etb)
        for k in range(8):
            sl = pl.ds(k * 16, 16)
            segb[sl] = dstb[sl] * RELS + etb[sl]
        pltpu.sync_copy(onesb, cnt_sp.at[segb], add=True)
        return 0
    lax.fori_loop(0, EPAD // NS // CH, cnt_chunk, 0)
    plsc.subcore_barrier()

    pltpu.sync_copy(cnt_sp.at[pl.ds(s * (CNTP // NS), CNTP // NS)], zb1)

    def inv_loop(i, _):
        sl = pl.ds(i * 16, 16)
        zb1[sl] = 1.0 / jnp.maximum(zb1[sl], 1.0)
        return 0
    lax.fori_loop(0, (CNTP // NS) // 16, inv_loop, 0)
    pltpu.sync_copy(zb1, cnt_sp.at[pl.ds(s * (CNTP // NS), CNTP // NS)])
    plsc.subcore_barrier()

    def agg_chunk(j, _):
        b = (c * NS + s) * (EPAD // 32) + j * CH
        pltpu.sync_copy(src_hbm.at[pl.ds(b, CH)], srcb)
        pltpu.sync_copy(dsts_hbm.at[pl.ds(b, CH)], dstb)
        pltpu.sync_copy(et_hbm.at[pl.ds(b, CH)], etb)
        for k in range(8):
            sl = pl.ds(k * 16, 16)
            gidxb[sl] = etb[sl] * N + srcb[sl]
            segb[sl] = dstb[sl] * RELS + etb[sl]
        pltpu.async_copy(xw_hbm.at[gidxb], rows, sem).wait()
        pltpu.async_copy(cnt_sp.at[segb], invw, sem).wait()

        def escale(g, _):
            iw16 = invw[pl.ds(g * 16, 16)]
            for e2 in range(16):
                w_e = iw16[e2]
                row = g * 16 + e2
                for k in range(8):
                    sl = pl.ds(k * 16, 16)
                    rows[row, sl] = rows[row, sl] * w_e
            return 0
        lax.fori_loop(0, CH // 16, escale, 0)
        pltpu.sync_copy(rows, agg_sp.at[dstb], add=True)
        return 0
    lax.fori_loop(0, EPAD // 32 // CH, agg_chunk, 0)
    plsc.subcore_barrier()

    for j in range(NPAD // NS // CH):
        r0 = s * (NPAD // NS) + j * CH
        pltpu.sync_copy(agg_sp.at[pl.ds(r0, CH)], rows)
        pltpu.sync_copy(rows, out_hbm.at[c, pl.ds(r0, CH)])


@functools.partial(
    pl.kernel,
    out_type=(jax.ShapeDtypeStruct((2, NPAD, DH), jnp.float32),
              jax.ShapeDtypeStruct((2, 2, NPAD), jnp.float32),
              jax.ShapeDtypeStruct((2, 2, EPAD), jnp.float32)),
    mesh=_SC_MESH,
    scratch_types=[
        pltpu.VMEM((GRP,), jnp.int32),
        pltpu.VMEM((GRP,), jnp.int32),
        pltpu.VMEM((GRP,), jnp.float32),
        pltpu.VMEM((GRP,), jnp.float32),
        pltpu.VMEM((CHA, DH), jnp.float32),
        pltpu.VMEM((CHA, DH), jnp.float32),
        pltpu.VMEM((CHA, DH), jnp.float32),
        pltpu.VMEM((CHA, DH), jnp.float32),
        pltpu.VMEM((CHA,), jnp.float32),
        pltpu.VMEM((CHA,), jnp.float32),
        pltpu.VMEM((CHA,), jnp.int32),
        pltpu.VMEM((16,), jnp.float32),
        pltpu.VMEM((NS, 16), jnp.float32),
        pltpu.VMEM((NPAD // NS,), jnp.float32),
        pltpu.VMEM((16, 32), jnp.float32),
        pltpu.VMEM((16, 32), jnp.float32),
        pltpu.VMEM_SHARED((NPAD, DH), jnp.float32),
        pltpu.VMEM_SHARED((NPAD,), jnp.float32),
        pltpu.VMEM_SHARED((NPAD,), jnp.float32),
        pltpu.VMEM_SHARED((NS, 16), jnp.float32),
        pltpu.SemaphoreType.DMA,
        pltpu.SemaphoreType.DMA,
        pltpu.SemaphoreType.DMA,
        pltpu.SemaphoreType.DMA,
    ],
)
def _attn_sc(src_hbm, dstg_hbm, dsts_hbm,
             q01, q23, k01, k23, v01, v23,
             num_out, asum_out, alpha_out,
             g0, g1, albuf0, albuf1, qr0, qr1, kr0, kr1, aexp0, aexp1,
             dstsb, mbuf, mall, zb, pad0, pad1,
             num_sp, as0_sp, as1_sp, mx_sp, sq0, sq1, sk0, sk1):
    c = lax.axis_index("c")
    s = lax.axis_index("s")
    tbase = s * (EPAD // NS)
    qbufs = (qr0, qr1)
    kbufs = (kr0, kr1)
    sqs = (sq0, sq1)
    sks = (sk0, sk1)

    def lanesum(v, pad, slot):
        for sh in (8, 4, 2, 1):
            pad[slot, pl.ds(0, 16)] = v
            v = v + pad[slot, pl.ds(sh, 16)]
        return v[0]

    def zrow(e, _):
        for k in range(8):
            qr0[e, pl.ds(k * 16, 16)] = _Z16()
        return 0
    lax.fori_loop(0, CHA, zrow, 0)

    def zzb(i, _):
        zb[pl.ds(i * 16, 16)] = _Z16()
        return 0
    lax.fori_loop(0, (NPAD // NS) // 16, zzb, 0)
    for r in range(16):
        pad0[r, pl.ds(0, 16)] = _Z16()
        pad0[r, pl.ds(16, 16)] = _Z16()
        pad1[r, pl.ds(0, 16)] = _Z16()
        pad1[r, pl.ds(16, 16)] = _Z16()

    for j in range(NPAD // NS // CHA):
        pltpu.sync_copy(qr0, num_sp.at[pl.ds(s * (NPAD // NS) + j * CHA, CHA)])
    pltpu.sync_copy(zb, as0_sp.at[pl.ds(s * (NPAD // NS), NPAD // NS)])
    pltpu.sync_copy(zb, as1_sp.at[pl.ds(s * (NPAD // NS), NPAD // NS)])
    plsc.subcore_barrier()

    def phase_a(qt, kt):
        def issue(jj, b):
            pltpu.async_copy(qt.at[g1.at[pl.ds(jj * CHA, CHA)]],
                             qbufs[b], sqs[b])
            pltpu.async_copy(kt.at[g0.at[pl.ds(jj * CHA, CHA)]],
                             kbufs[b], sks[b])

        def waitb(b):
            pltpu.make_async_copy(qt.at[pl.ds(0, CHA)], qbufs[b],
                                  sqs[b]).wait()
            pltpu.make_async_copy(kt.at[pl.ds(0, CHA)], kbufs[b],
                                  sks[b]).wait()

        lane = lax.iota(jnp.int32, 16)

        def compute(jj, b, m):
            qb = qbufs[b]
            kb = kbufs[b]

            def egrp(g, m):
                z0 = _Z16()
                z1 = _Z16()
                for e2 in range(16):
                    row = g * 16 + e2
                    p0 = qb[row, pl.ds(0, 16)] * kb[row, pl.ds(0, 16)]
                    for k in range(1, 4):
                        sl = pl.ds(k * 16, 16)
                        p0 = p0 + qb[row, sl] * kb[row, sl]
                    p1 = qb[row, pl.ds(64, 16)] * kb[row, pl.ds(64, 16)]
                    for k in range(5, 8):
                        sl = pl.ds(k * 16, 16)
                        p1 = p1 + qb[row, sl] * kb[row, sl]
                    s0 = lanesum(p0, pad0, e2)
                    s1 = lanesum(p1, pad1, e2)
                    z0 = jnp.where(lane == e2, s0, z0)
                    z1 = jnp.where(lane == e2, s1, z1)
                albuf0[pl.ds(jj * CHA + g * 16, 16)] = z0
                albuf1[pl.ds(jj * CHA + g * 16, 16)] = z1
                return jnp.maximum(m, jnp.maximum(z0, z1))
            return lax.fori_loop(0, CHA // 16, egrp, m)

        def pair(p, m):
            issue(2 * p + 1, 1)
            waitb(0)
            m = compute(2 * p, 0, m)

            @pl.when(p < (GRP // CHA) // 2 - 1)
            def _():
                issue(2 * p + 2, 0)
            waitb(1)
            return compute(2 * p + 1, 1, m)

        def grp_loop(gi, m):
            gb = tbase + gi * GRP
            pltpu.sync_copy(src_hbm.at[pl.ds(gb, GRP)], g0)
            pltpu.sync_copy(dstg_hbm.at[pl.ds(gb, GRP)], g1)
            issue(0, 0)
            m = lax.fori_loop(0, (GRP // CHA) // 2, pair, m)
            pltpu.sync_copy(albuf0, alpha_out.at[c, 0, pl.ds(gb, GRP)])
            pltpu.sync_copy(albuf1, alpha_out.at[c, 1, pl.ds(gb, GRP)])
            return m
        m = lax.fori_loop(0, EPAD // NS // GRP, grp_loop,
                          jnp.full((16,), -3.0e38, jnp.float32))
        mbuf[pl.ds(0, 16)] = m

    @pl.when(c == 0)
    def _():
        phase_a(q01, k01)

    @pl.when(c == 1)
    def _():
        phase_a(q23, k23)

    pltpu.sync_copy(mbuf, mx_sp.at[s])
    plsc.subcore_barrier()
    pltpu.sync_copy(mx_sp, mall)
    gv = mall[0, pl.ds(0, 16)]
    for t in range(1, NS):
        gv = jnp.maximum(gv, mall[t, pl.ds(0, 16)])
    gmax = gv[0]
    for t in range(1, 16):
        gmax = jnp.maximum(gmax, gv[t])

    def phase_b(vt):
        def issuev(jj, b):
            pltpu.async_copy(vt.at[g0.at[pl.ds(jj * CHA, CHA)]],
                             qbufs[b], sqs[b])

        def waitv(b):
            pltpu.make_async_copy(vt.at[pl.ds(0, CHA)], qbufs[b],
                                  sqs[b]).wait()

        def computeb(jj, b):
            vb = qbufs[b]
            for k in range(CHA // 16):
                sl = pl.ds(k * 16, 16)
                asl = pl.ds(jj * CHA + k * 16, 16)
                aexp0[sl] = jnp.exp(albuf0[asl] - gmax)
                aexp1[sl] = jnp.exp(albuf1[asl] - gmax)
                dstsb[sl] = g1[asl]

            def escale(g, _):
                a0v = aexp0[pl.ds(g * 16, 16)]
                a1v = aexp1[pl.ds(g * 16, 16)]
                for e2 in range(16):
                    row = g * 16 + e2
                    a0 = a0v[e2]
                    a1 = a1v[e2]
                    for k in range(4):
                        sl = pl.ds(k * 16, 16)
                        vb[row, sl] = vb[row, sl] * a0
                    for k in range(4, 8):
                        sl = pl.ds(k * 16, 16)
                        vb[row, sl] = vb[row, sl] * a1
                return 0
            lax.fori_loop(0, CHA // 16, escale, 0)
            pltpu.sync_copy(aexp0, as0_sp.at[dstsb], add=True)
            pltpu.sync_copy(aexp1, as1_sp.at[dstsb], add=True)
            pltpu.sync_copy(vb, num_sp.at[dstsb], add=True)

        def pairb(p, _):
            issuev(2 * p + 1, 1)
            waitv(0)
            computeb(2 * p, 0)

            @pl.when(p < (GRP // CHA) // 2 - 1)
            def _():
                issuev(2 * p + 2, 0)
            waitv(1)
            computeb(2 * p + 1, 1)
            return 0

        def grpb(gi, _):
            gb = tbase + gi * GRP
            pltpu.sync_copy(src_hbm.at[pl.ds(gb, GRP)], g0)
            pltpu.sync_copy(dsts_hbm.at[pl.ds(gb, GRP)], g1)
            pltpu.sync_copy(alpha_out.at[c, 0, pl.ds(gb, GRP)], albuf0)
            pltpu.sync_copy(alpha_out.at[c, 1, pl.ds(gb, GRP)], albuf1)
            issuev(0, 0)
            lax.fori_loop(0, (GRP // CHA) // 2, pairb, 0)
            return 0
        lax.fori_loop(0, EPAD // NS // GRP, grpb, 0)

    @pl.when(c == 0)
    def _():
        phase_b(v01)

    @pl.when(c == 1)
    def _():
        phase_b(v23)

    plsc.subcore_barrier()

    for j in range(NPAD // NS // CHA):
        r0 = s * (NPAD // NS) + j * CHA
        pltpu.sync_copy(num_sp.at[pl.ds(r0, CHA)], kr0)
        pltpu.sync_copy(kr0, num_out.at[c, pl.ds(r0, CHA)])
    a0 = s * (NPAD // NS)
    pltpu.sync_copy(as0_sp.at[pl.ds(a0, NPAD // NS)], zb)
    pltpu.sync_copy(zb, asum_out.at[c, 0, pl.ds(a0, NPAD // NS)])
    pltpu.sync_copy(as1_sp.at[pl.ds(a0, NPAD // NS)], zb)
    pltpu.sync_copy(zb, asum_out.at[c, 1, pl.ds(a0, NPAD // NS)])


def kernel(x, edge_index, edge_type, rgcn_w, rgcn_root, rgcn_bias,
           wq, bq, wk, bk, wv, bv, wskip, bskip, fc_w, fc_b):
    src = edge_index[0].astype(jnp.int32)
    dst = edge_index[1].astype(jnp.int32)
    et = edge_type.astype(jnp.int32)

    pcnt = EPAD - E
    pidx = jnp.arange(pcnt, dtype=jnp.int32)
    src_g = jnp.concatenate([src, (pidx * 997) % N])
    dst_g = jnp.concatenate([dst, (pidx * 1013 + 7) % N])
    dst_s = jnp.concatenate([dst, N + (pidx % 16)])
    et_g = jnp.concatenate([et, jnp.zeros((pcnt,), jnp.int32)])

    w7 = jnp.concatenate([rgcn_w, rgcn_root[None]], axis=0)
    xw7 = pl.pallas_call(
        _mm7_body,
        grid=(N // NB, 7),
        in_specs=[pl.BlockSpec((NB, 768), lambda i, r: (i, 0)),
                  pl.BlockSpec((1, 768, DH), lambda i, r: (r, 0, 0))],
        out_specs=pl.BlockSpec((1, NB, DH), lambda i, r: (r, i, 0)),
        out_shape=jax.ShapeDtypeStruct((7, N, DH), jnp.float32),
    )(x, w7)
    xw_flat = xw7[:RELS].reshape(RELS * N, DH)
    xroot = xw7[RELS]

    agg2 = _rgcn_sc(src_g, dst_s, et_g, xw_flat)

    wcat = jnp.concatenate([wq * 0.125, wk, wv, wskip], axis=1)
    bcat = jnp.concatenate([bq * 0.125, bk, bv, bskip])[None]
    outs = pl.pallas_call(
        _qkv_body,
        grid=(N // NB,),
        in_specs=[pl.BlockSpec((2, NB, DH), lambda i: (0, i, 0)),
                  pl.BlockSpec((NB, DH), lambda i: (i, 0)),
                  pl.BlockSpec((1, DH), lambda i: (0, 0)),
                  pl.BlockSpec((DH, 1024), lambda i: (0, 0)),
                  pl.BlockSpec((1, 1024), lambda i: (0, 0))],
        out_specs=[pl.BlockSpec((NB, DH), lambda i: (i, 0))] * 6
        + [pl.BlockSpec((NB, 256), lambda i: (i, 0))],
        out_shape=[jax.ShapeDtypeStruct((N, DH), jnp.float32)] * 6
        + [jax.ShapeDtypeStruct((N, 256), jnp.float32)],
    )(agg2, xroot, rgcn_bias[None], wcat, bcat)
    q01, q23, k01, k23, v01, v23, skip = outs

    num2, asum2, _ = _attn_sc(src_g, dst_g, dst_s, q01, q23, k01, k23,
                              v01, v23)

    skip_p = jnp.pad(skip, ((0, NPAD - N), (0, 0)))
    fcw_p = jnp.pad(fc_w, ((0, 0), (0, 125)))
    fcb_p = jnp.pad(fc_b, (0, 125))[None]
    out = pl.pallas_call(
        _final_body,
        grid=(NPAD // 1024,),
        in_specs=[pl.BlockSpec((2, 1024, DH), lambda i: (0, i, 0)),
                  pl.BlockSpec((2, 2, 1024), lambda i: (0, 0, i)),
                  pl.BlockSpec((1024, 256), lambda i: (i, 0)),
                  pl.BlockSpec((1, 256), lambda i: (0, 0)),
                  pl.BlockSpec((256, 128), lambda i: (0, 0)),
                  pl.BlockSpec((1, 128), lambda i: (0, 0))],
        out_specs=pl.BlockSpec((1, 128), lambda i: (0, 0)),
        out_shape=jax.ShapeDtypeStruct((1, 128), jnp.float32),
        scratch_shapes=[pltpu.VMEM((1, 256), jnp.float32)],
    )(num2, asum2, skip_p, bskip[None], fcw_p, fcb_p)
    return out[:, :3]


# speedup vs baseline: 25.3651x; 1.1696x over previous
import functools

import jax
import jax.numpy as jnp
from jax import lax
from jax.experimental import pallas as pl
from jax.experimental.pallas import tpu as pltpu
from jax.experimental.pallas import tpu_sc as plsc

N = 10000
E = 160000
RELS = 6
DH = 128
NPAD = 10240
EPAD = 163840
CH = 128
CNTP = 60416
NS = 16
NB = 1000
CHA = 64
GRP = 1024


def _mm7_body(x_ref, w_ref, o_ref):
    o_ref[0] = jnp.dot(x_ref[...], w_ref[0], preferred_element_type=jnp.float32)


def _qkv_body(agg_ref, xroot_ref, bias_ref, wcat_ref, bcat_ref,
              q01_ref, q23_ref, k01_ref, k23_ref, v01_ref, v23_ref, skip_ref):
    h = agg_ref[0] + agg_ref[1] + xroot_ref[...] + bias_ref[...]
    h = jnp.maximum(h, 0.0)
    y = jnp.dot(h, wcat_ref[...], preferred_element_type=jnp.float32) + bcat_ref[...]
    q01_ref[...] = y[:, 0:128]
    q23_ref[...] = y[:, 128:256]
    k01_ref[...] = y[:, 256:384]
    k23_ref[...] = y[:, 384:512]
    v01_ref[...] = y[:, 512:640]
    v23_ref[...] = y[:, 640:768]
    skip_ref[...] = y[:, 768:1024]


def _final_body(num_ref, asum_ref, skip_ref, bskip_ref, fcw_ref, fcb_ref,
                o_ref, acc_ref):
    i = pl.program_id(0)
    blk = 1024
    a = asum_ref[...]
    a00 = jnp.maximum(a[0, 0, :], 1e-30).reshape(blk, 1)
    a01 = jnp.maximum(a[0, 1, :], 1e-30).reshape(blk, 1)
    a10 = jnp.maximum(a[1, 0, :], 1e-30).reshape(blk, 1)
    a11 = jnp.maximum(a[1, 1, :], 1e-30).reshape(blk, 1)
    nm = num_ref[...]
    o = jnp.concatenate(
        [nm[0, :, 0:64] / a00, nm[0, :, 64:128] / a01,
         nm[1, :, 0:64] / a10, nm[1, :, 64:128] / a11], axis=1)
    o = jnp.maximum(o + skip_ref[...] + bskip_ref[...], 0.0)
    rows = i * blk + lax.broadcasted_iota(jnp.int32, (blk, 1), 0)
    o = jnp.where(rows < N, o, 0.0)
    part = jnp.sum(o, axis=0, keepdims=True)

    @pl.when(i == 0)
    def _():
        acc_ref[...] = part

    @pl.when(i > 0)
    def _():
        acc_ref[...] += part

    @pl.when(i == (NPAD // blk) - 1)
    def _():
        o_ref[...] = jnp.dot(acc_ref[...] * (1.0 / N), fcw_ref[...],
                             preferred_element_type=jnp.float32) + fcb_ref[...]


_SC_MESH = plsc.VectorSubcoreMesh(core_axis_name="c", subcore_axis_name="s")
_Z16 = functools.partial(jnp.zeros, (16,), jnp.float32)


@functools.partial(
    pl.kernel,
    out_type=jax.ShapeDtypeStruct((2, NPAD, DH), jnp.float32),
    mesh=_SC_MESH,
    scratch_types=[
        pltpu.VMEM((GRP,), jnp.int32),
        pltpu.VMEM((GRP,), jnp.int32),
        pltpu.VMEM((GRP,), jnp.int32),
        pltpu.VMEM((CH,), jnp.int32),
        pltpu.VMEM((CH,), jnp.int32),
        pltpu.VMEM((CH,), jnp.int32),
        pltpu.VMEM((CH,), jnp.int32),
        pltpu.VMEM((CH,), jnp.int32),
        pltpu.VMEM((CH,), jnp.int32),
        pltpu.VMEM((CH,), jnp.float32),
        pltpu.VMEM((CH, DH), jnp.float32),
        pltpu.VMEM((CH, DH), jnp.float32),
        pltpu.VMEM((CH,), jnp.float32),
        pltpu.VMEM((CH,), jnp.float32),
        pltpu.VMEM((CNTP // NS,), jnp.float32),
        pltpu.VMEM_SHARED((NPAD, DH), jnp.float32),
        pltpu.VMEM_SHARED((CNTP,), jnp.float32),
        pltpu.SemaphoreType.DMA,
        pltpu.SemaphoreType.DMA,
        pltpu.SemaphoreType.DMA,
        pltpu.SemaphoreType.DMA,
    ],
)
def _rgcn_sc(src_hbm, dsts_hbm, et_hbm, xw_hbm, out_hbm,
             gs, gd, ge, segb0, segb1, gidx0, gidx1, dstb0, dstb1,
             onesb, rows0, rows1, invw0, invw1, zb1,
             agg_sp, cnt_sp, sr0, sr1, si0, si1):
    c = lax.axis_index("c")
    s = lax.axis_index("s")
    rowsb = (rows0, rows1)
    segb = (segb0, segb1)
    gidxb = (gidx0, gidx1)
    dstb = (dstb0, dstb1)
    invwb = (invw0, invw1)
    srs = (sr0, sr1)
    sis = (si0, si1)

    def zrow(e, _):
        for k in range(8):
            rows0[e, pl.ds(k * 16, 16)] = _Z16()
        return 0
    lax.fori_loop(0, CH, zrow, 0)

    def zzb(i, _):
        zb1[pl.ds(i * 16, 16)] = _Z16()
        return 0
    lax.fori_loop(0, (CNTP // NS) // 16, zzb, 0)

    for j in range(NPAD // NS // CH):
        pltpu.sync_copy(rows0, agg_sp.at[pl.ds(s * (NPAD // NS) + j * CH, CH)])
    pltpu.sync_copy(zb1, cnt_sp.at[pl.ds(s * (CNTP // NS), CNTP // NS)])
    for k in range(8):
        onesb[pl.ds(k * 16, 16)] = _Z16() + 1.0
    plsc.subcore_barrier()

    def cnt_grp(gi, _):
        gb = s * (EPAD // NS) + gi * GRP
        pltpu.sync_copy(dsts_hbm.at[pl.ds(gb, GRP)], gd)
        pltpu.sync_copy(et_hbm.at[pl.ds(gb, GRP)], ge)

        def cchunk(j, _):
            for k in range(8):
                sl = pl.ds(k * 16, 16)
                asl = pl.ds(j * CH + k * 16, 16)
                segb0[sl] = gd[asl] * RELS + ge[asl]
            pltpu.sync_copy(onesb, cnt_sp.at[segb0], add=True)
            return 0
        lax.fori_loop(0, GRP // CH, cchunk, 0)
        return 0
    lax.fori_loop(0, EPAD // NS // GRP, cnt_grp, 0)
    plsc.subcore_barrier()

    pltpu.sync_copy(cnt_sp.at[pl.ds(s * (CNTP // NS), CNTP // NS)], zb1)

    def inv_loop(i, _):
        sl = pl.ds(i * 16, 16)
        zb1[sl] = 1.0 / jnp.maximum(zb1[sl], 1.0)
        return 0
    lax.fori_loop(0, (CNTP // NS) // 16, inv_loop, 0)
    pltpu.sync_copy(zb1, cnt_sp.at[pl.ds(s * (CNTP // NS), CNTP // NS)])
    plsc.subcore_barrier()

    def issue(j, b):
        for k in range(8):
            sl = pl.ds(k * 16, 16)
            asl = pl.ds(j * CH + k * 16, 16)
            gidxb[b][sl] = ge[asl] * N + gs[asl]
            segb[b][sl] = gd[asl] * RELS + ge[asl]
            dstb[b][sl] = gd[asl]
        pltpu.async_copy(xw_hbm.at[gidxb[b]], rowsb[b], srs[b])
        pltpu.async_copy(cnt_sp.at[segb[b]], invwb[b], sis[b])

    def waitb(b):
        pltpu.make_async_copy(xw_hbm.at[pl.ds(0, CH)], rowsb[b],
                              srs[b]).wait()
        pltpu.make_async_copy(xw_hbm.at[0, pl.ds(0, CH)], invwb[b],
                              sis[b]).wait()

    def compute(b):
        rows = rowsb[b]
        invw = invwb[b]

        def escale(g, _):
            iw16 = invw[pl.ds(g * 16, 16)]
            for e2 in range(16):
                w_e = iw16[e2]
                row = g * 16 + e2
                for k in range(8):
                    sl = pl.ds(k * 16, 16)
                    rows[row, sl] = rows[row, sl] * w_e
            return 0
        lax.fori_loop(0, CH // 16, escale, 0)
        pltpu.sync_copy(rows, agg_sp.at[dstb[b]], add=True)

    def pair(p, _):
        issue(2 * p + 1, 1)
        waitb(0)
        compute(0)

        @pl.when(p < (GRP // CH) // 2 - 1)
        def _():
            issue(2 * p + 2, 0)
        waitb(1)
        compute(1)
        return 0

    def agg_grp(gi, _):
        gb = (c * NS + s) * (EPAD // 32) + gi * GRP
        pltpu.sync_copy(src_hbm.at[pl.ds(gb, GRP)], gs)
        pltpu.sync_copy(dsts_hbm.at[pl.ds(gb, GRP)], gd)
        pltpu.sync_copy(et_hbm.at[pl.ds(gb, GRP)], ge)
        issue(0, 0)
        lax.fori_loop(0, (GRP // CH) // 2, pair, 0)
        return 0
    lax.fori_loop(0, EPAD // 32 // GRP, agg_grp, 0)
    plsc.subcore_barrier()

    for j in range(NPAD // NS // CH):
        r0 = s * (NPAD // NS) + j * CH
        pltpu.sync_copy(agg_sp.at[pl.ds(r0, CH)], rows0)
        pltpu.sync_copy(rows0, out_hbm.at[c, pl.ds(r0, CH)])


@functools.partial(
    pl.kernel,
    out_type=(jax.ShapeDtypeStruct((2, NPAD, DH), jnp.float32),
              jax.ShapeDtypeStruct((2, 2, NPAD), jnp.float32),
              jax.ShapeDtypeStruct((2, 2, EPAD), jnp.float32)),
    mesh=_SC_MESH,
    scratch_types=[
        pltpu.VMEM((GRP,), jnp.int32),
        pltpu.VMEM((GRP,), jnp.int32),
        pltpu.VMEM((GRP,), jnp.float32),
        pltpu.VMEM((GRP,), jnp.float32),
        pltpu.VMEM((CHA, DH), jnp.float32),
        pltpu.VMEM((CHA, DH), jnp.float32),
        pltpu.VMEM((CHA, DH), jnp.float32),
        pltpu.VMEM((CHA, DH), jnp.float32),
        pltpu.VMEM((CHA,), jnp.float32),
        pltpu.VMEM((CHA,), jnp.float32),
        pltpu.VMEM((CHA,), jnp.int32),
        pltpu.VMEM((16,), jnp.float32),
        pltpu.VMEM((NS, 16), jnp.float32),
        pltpu.VMEM((NPAD // NS,), jnp.float32),
        pltpu.VMEM((16, 32), jnp.float32),
        pltpu.VMEM((16, 32), jnp.float32),
        pltpu.VMEM_SHARED((NPAD, DH), jnp.float32),
        pltpu.VMEM_SHARED((NPAD,), jnp.float32),
        pltpu.VMEM_SHARED((NPAD,), jnp.float32),
        pltpu.VMEM_SHARED((NS, 16), jnp.float32),
        pltpu.SemaphoreType.DMA,
        pltpu.SemaphoreType.DMA,
        pltpu.SemaphoreType.DMA,
        pltpu.SemaphoreType.DMA,
    ],
)
def _attn_sc(src_hbm, dstg_hbm, dsts_hbm,
             q01, q23, k01, k23, v01, v23,
             num_out, asum_out, alpha_out,
             g0, g1, albuf0, albuf1, qr0, qr1, kr0, kr1, aexp0, aexp1,
             dstsb, mbuf, mall, zb, pad0, pad1,
             num_sp, as0_sp, as1_sp, mx_sp, sq0, sq1, sk0, sk1):
    c = lax.axis_index("c")
    s = lax.axis_index("s")
    tbase = s * (EPAD // NS)
    qbufs = (qr0, qr1)
    kbufs = (kr0, kr1)
    sqs = (sq0, sq1)
    sks = (sk0, sk1)

    def lanesum(v, pad, slot):
        for sh in (8, 4, 2, 1):
            pad[slot, pl.ds(0, 16)] = v
            v = v + pad[slot, pl.ds(sh, 16)]
        return v[0]

    def zrow(e, _):
        for k in range(8):
            qr0[e, pl.ds(k * 16, 16)] = _Z16()
        return 0
    lax.fori_loop(0, CHA, zrow, 0)

    def zzb(i, _):
        zb[pl.ds(i * 16, 16)] = _Z16()
        return 0
    lax.fori_loop(0, (NPAD // NS) // 16, zzb, 0)
    for r in range(16):
        pad0[r, pl.ds(0, 16)] = _Z16()
        pad0[r, pl.ds(16, 16)] = _Z16()
        pad1[r, pl.ds(0, 16)] = _Z16()
        pad1[r, pl.ds(16, 16)] = _Z16()

    for j in range(NPAD // NS // CHA):
        pltpu.sync_copy(qr0, num_sp.at[pl.ds(s * (NPAD // NS) + j * CHA, CHA)])
    pltpu.sync_copy(zb, as0_sp.at[pl.ds(s * (NPAD // NS), NPAD // NS)])
    pltpu.sync_copy(zb, as1_sp.at[pl.ds(s * (NPAD // NS), NPAD // NS)])
    plsc.subcore_barrier()

    def phase_a(qt, kt):
        def issue(jj, b):
            pltpu.async_copy(qt.at[g1.at[pl.ds(jj * CHA, CHA)]],
                             qbufs[b], sqs[b])
            pltpu.async_copy(kt.at[g0.at[pl.ds(jj * CHA, CHA)]],
                             kbufs[b], sks[b])

        def waitb(b):
            pltpu.make_async_copy(qt.at[pl.ds(0, CHA)], qbufs[b],
                                  sqs[b]).wait()
            pltpu.make_async_copy(kt.at[pl.ds(0, CHA)], kbufs[b],
                                  sks[b]).wait()

        lane = lax.iota(jnp.int32, 16)

        def compute(jj, b, m):
            qb = qbufs[b]
            kb = kbufs[b]

            def egrp(g, m):
                z0 = _Z16()
                z1 = _Z16()
                for e2 in range(16):
                    row = g * 16 + e2
                    p0 = qb[row, pl.ds(0, 16)] * kb[row, pl.ds(0, 16)]
                    for k in range(1, 4):
                        sl = pl.ds(k * 16, 16)
                        p0 = p0 + qb[row, sl] * kb[row, sl]
                    p1 = qb[row, pl.ds(64, 16)] * kb[row, pl.ds(64, 16)]
                    for k in range(5, 8):
                        sl = pl.ds(k * 16, 16)
                        p1 = p1 + qb[row, sl] * kb[row, sl]
                    s0 = lanesum(p0, pad0, e2)
                    s1 = lanesum(p1, pad1, e2)
                    z0 = jnp.where(lane == e2, s0, z0)
                    z1 = jnp.where(lane == e2, s1, z1)
                albuf0[pl.ds(jj * CHA + g * 16, 16)] = z0
                albuf1[pl.ds(jj * CHA + g * 16, 16)] = z1
                return jnp.maximum(m, jnp.maximum(z0, z1))
            return lax.fori_loop(0, CHA // 16, egrp, m)

        def pair(p, m):
            issue(2 * p + 1, 1)
            waitb(0)
            m = compute(2 * p, 0, m)

            @pl.when(p < (GRP // CHA) // 2 - 1)
            def _():
                issue(2 * p + 2, 0)
            waitb(1)
            return compute(2 * p + 1, 1, m)

        def grp_loop(gi, m):
            gb = tbase + gi * GRP
            pltpu.sync_copy(src_hbm.at[pl.ds(gb, GRP)], g0)
            pltpu.sync_copy(dstg_hbm.at[pl.ds(gb, GRP)], g1)
            issue(0, 0)
            m = lax.fori_loop(0, (GRP // CHA) // 2, pair, m)
            pltpu.sync_copy(albuf0, alpha_out.at[c, 0, pl.ds(gb, GRP)])
            pltpu.sync_copy(albuf1, alpha_out.at[c, 1, pl.ds(gb, GRP)])
            return m
        m = lax.fori_loop(0, EPAD // NS // GRP, grp_loop,
                          jnp.full((16,), -3.0e38, jnp.float32))
        mbuf[pl.ds(0, 16)] = m

    @pl.when(c == 0)
    def _():
        phase_a(q01, k01)

    @pl.when(c == 1)
    def _():
        phase_a(q23, k23)

    pltpu.sync_copy(mbuf, mx_sp.at[s])
    plsc.subcore_barrier()
    pltpu.sync_copy(mx_sp, mall)
    gv = mall[0, pl.ds(0, 16)]
    for t in range(1, NS):
        gv = jnp.maximum(gv, mall[t, pl.ds(0, 16)])
    gmax = gv[0]
    for t in range(1, 16):
        gmax = jnp.maximum(gmax, gv[t])

    def phase_b(vt):
        def issuev(jj, b):
            pltpu.async_copy(vt.at[g0.at[pl.ds(jj * CHA, CHA)]],
                             qbufs[b], sqs[b])

        def waitv(b):
            pltpu.make_async_copy(vt.at[pl.ds(0, CHA)], qbufs[b],
                                  sqs[b]).wait()

        def computeb(jj, b):
            vb = qbufs[b]
            for k in range(CHA // 16):
                sl = pl.ds(k * 16, 16)
                asl = pl.ds(jj * CHA + k * 16, 16)
                aexp0[sl] = jnp.exp(albuf0[asl] - gmax)
                aexp1[sl] = jnp.exp(albuf1[asl] - gmax)
                dstsb[sl] = g1[asl]

            def escale(g, _):
                a0v = aexp0[pl.ds(g * 16, 16)]
                a1v = aexp1[pl.ds(g * 16, 16)]
                for e2 in range(16):
                    row = g * 16 + e2
                    a0 = a0v[e2]
                    a1 = a1v[e2]
                    for k in range(4):
                        sl = pl.ds(k * 16, 16)
                        vb[row, sl] = vb[row, sl] * a0
                    for k in range(4, 8):
                        sl = pl.ds(k * 16, 16)
                        vb[row, sl] = vb[row, sl] * a1
                return 0
            lax.fori_loop(0, CHA // 16, escale, 0)
            pltpu.sync_copy(aexp0, as0_sp.at[dstsb], add=True)
            pltpu.sync_copy(aexp1, as1_sp.at[dstsb], add=True)
            pltpu.sync_copy(vb, num_sp.at[dstsb], add=True)

        def pairb(p, _):
            issuev(2 * p + 1, 1)
            waitv(0)
            computeb(2 * p, 0)

            @pl.when(p < (GRP // CHA) // 2 - 1)
            def _():
                issuev(2 * p + 2, 0)
            waitv(1)
            computeb(2 * p + 1, 1)
            return 0

        def grpb(gi, _):
            gb = tbase + gi * GRP
            pltpu.sync_copy(src_hbm.at[pl.ds(gb, GRP)], g0)
            pltpu.sync_copy(dsts_hbm.at[pl.ds(gb, GRP)], g1)
            pltpu.sync_copy(alpha_out.at[c, 0, pl.ds(gb, GRP)], albuf0)
            pltpu.sync_copy(alpha_out.at[c, 1, pl.ds(gb, GRP)], albuf1)
            issuev(0, 0)
            lax.fori_loop(0, (GRP // CHA) // 2, pairb, 0)
            return 0
        lax.fori_loop(0, EPAD // NS // GRP, grpb, 0)

    @pl.when(c == 0)
    def _():
        phase_b(v01)

    @pl.when(c == 1)
    def _():
        phase_b(v23)

    plsc.subcore_barrier()

    for j in range(NPAD // NS // CHA):
        r0 = s * (NPAD // NS) + j * CHA
        pltpu.sync_copy(num_sp.at[pl.ds(r0, CHA)], kr0)
        pltpu.sync_copy(kr0, num_out.at[c, pl.ds(r0, CHA)])
    a0 = s * (NPAD // NS)
    pltpu.sync_copy(as0_sp.at[pl.ds(a0, NPAD // NS)], zb)
    pltpu.sync_copy(zb, asum_out.at[c, 0, pl.ds(a0, NPAD // NS)])
    pltpu.sync_copy(as1_sp.at[pl.ds(a0, NPAD // NS)], zb)
    pltpu.sync_copy(zb, asum_out.at[c, 1, pl.ds(a0, NPAD // NS)])


def kernel(x, edge_index, edge_type, rgcn_w, rgcn_root, rgcn_bias,
           wq, bq, wk, bk, wv, bv, wskip, bskip, fc_w, fc_b):
    src = edge_index[0].astype(jnp.int32)
    dst = edge_index[1].astype(jnp.int32)
    et = edge_type.astype(jnp.int32)

    pcnt = EPAD - E
    pidx = jnp.arange(pcnt, dtype=jnp.int32)
    src_g = jnp.concatenate([src, (pidx * 997) % N])
    dst_g = jnp.concatenate([dst, (pidx * 1013 + 7) % N])
    dst_s = jnp.concatenate([dst, N + (pidx % 16)])
    et_g = jnp.concatenate([et, jnp.zeros((pcnt,), jnp.int32)])

    w7 = jnp.concatenate([rgcn_w, rgcn_root[None]], axis=0)
    xw7 = pl.pallas_call(
        _mm7_body,
        grid=(N // NB, 7),
        in_specs=[pl.BlockSpec((NB, 768), lambda i, r: (i, 0)),
                  pl.BlockSpec((1, 768, DH), lambda i, r: (r, 0, 0))],
        out_specs=pl.BlockSpec((1, NB, DH), lambda i, r: (r, i, 0)),
        out_shape=jax.ShapeDtypeStruct((7, N, DH), jnp.float32),
    )(x, w7)
    xw_flat = xw7[:RELS].reshape(RELS * N, DH)
    xroot = xw7[RELS]

    agg2 = _rgcn_sc(src_g, dst_s, et_g, xw_flat)

    wcat = jnp.concatenate([wq * 0.125, wk, wv, wskip], axis=1)
    bcat = jnp.concatenate([bq * 0.125, bk, bv, bskip])[None]
    outs = pl.pallas_call(
        _qkv_body,
        grid=(N // NB,),
        in_specs=[pl.BlockSpec((2, NB, DH), lambda i: (0, i, 0)),
                  pl.BlockSpec((NB, DH), lambda i: (i, 0)),
                  pl.BlockSpec((1, DH), lambda i: (0, 0)),
                  pl.BlockSpec((DH, 1024), lambda i: (0, 0)),
                  pl.BlockSpec((1, 1024), lambda i: (0, 0))],
        out_specs=[pl.BlockSpec((NB, DH), lambda i: (i, 0))] * 6
        + [pl.BlockSpec((NB, 256), lambda i: (i, 0))],
        out_shape=[jax.ShapeDtypeStruct((N, DH), jnp.float32)] * 6
        + [jax.ShapeDtypeStruct((N, 256), jnp.float32)],
    )(agg2, xroot, rgcn_bias[None], wcat, bcat)
    q01, q23, k01, k23, v01, v23, skip = outs

    num2, asum2, _ = _attn_sc(src_g, dst_g, dst_s, q01, q23, k01, k23,
                              v01, v23)

    skip_p = jnp.pad(skip, ((0, NPAD - N), (0, 0)))
    fcw_p = jnp.pad(fc_w, ((0, 0), (0, 125)))
    fcb_p = jnp.pad(fc_b, (0, 125))[None]
    out = pl.pallas_call(
        _final_body,
        grid=(NPAD // 1024,),
        in_specs=[pl.BlockSpec((2, 1024, DH), lambda i: (0, i, 0)),
                  pl.BlockSpec((2, 2, 1024), lambda i: (0, 0, i)),
                  pl.BlockSpec((1024, 256), lambda i: (i, 0)),
                  pl.BlockSpec((1, 256), lambda i: (0, 0)),
                  pl.BlockSpec((256, 128), lambda i: (0, 0)),
                  pl.BlockSpec((1, 128), lambda i: (0, 0))],
        out_specs=pl.BlockSpec((1, 128), lambda i: (0, 0)),
        out_shape=jax.ShapeDtypeStruct((1, 128), jnp.float32),
        scratch_shapes=[pltpu.VMEM((1, 256), jnp.float32)],
    )(num2, asum2, skip_p, bskip[None], fcw_p, fcb_p)
    return out[:, :3]


# speedup vs baseline: 26.0801x; 1.0282x over previous
import functools

import jax
import jax.numpy as jnp
from jax import lax
from jax.experimental import pallas as pl
from jax.experimental.pallas import tpu as pltpu
from jax.experimental.pallas import tpu_sc as plsc

N = 10000
E = 160000
RELS = 6
DH = 128
NPAD = 10240
EPAD = 163840
CH = 128
CNTP = 60416
NS = 16
NB = 1000
CHA = 64
GRP = 1024


def _mm7_body(x_ref, w_ref, o_ref):
    o_ref[0] = jnp.dot(x_ref[...], w_ref[0], preferred_element_type=jnp.float32)


def _qkv_body(agg_ref, xroot_ref, bias_ref, wcat_ref, bcat_ref,
              q01_ref, q23_ref, k01_ref, k23_ref, v01_ref, v23_ref, skip_ref):
    h = agg_ref[0] + agg_ref[1] + xroot_ref[...] + bias_ref[...]
    h = jnp.maximum(h, 0.0)
    y = jnp.dot(h, wcat_ref[...], preferred_element_type=jnp.float32) + bcat_ref[...]
    q01_ref[...] = y[:, 0:128]
    q23_ref[...] = y[:, 128:256]
    k01_ref[...] = y[:, 256:384]
    k23_ref[...] = y[:, 384:512]
    v01_ref[...] = y[:, 512:640]
    v23_ref[...] = y[:, 640:768]
    skip_ref[...] = y[:, 768:1024]


def _final_body(num_ref, asum_ref, skip_ref, bskip_ref, fcw_ref, fcb_ref,
                o_ref, acc_ref):
    i = pl.program_id(0)
    blk = 1024
    a = asum_ref[...]
    a00 = jnp.maximum(a[0, 0, :], 1e-30).reshape(blk, 1)
    a01 = jnp.maximum(a[0, 1, :], 1e-30).reshape(blk, 1)
    a10 = jnp.maximum(a[1, 0, :], 1e-30).reshape(blk, 1)
    a11 = jnp.maximum(a[1, 1, :], 1e-30).reshape(blk, 1)
    nm = num_ref[...]
    o = jnp.concatenate(
        [nm[0, :, 0:64] / a00, nm[0, :, 64:128] / a01,
         nm[1, :, 0:64] / a10, nm[1, :, 64:128] / a11], axis=1)
    o = jnp.maximum(o + skip_ref[...] + bskip_ref[...], 0.0)
    rows = i * blk + lax.broadcasted_iota(jnp.int32, (blk, 1), 0)
    o = jnp.where(rows < N, o, 0.0)
    part = jnp.sum(o, axis=0, keepdims=True)

    @pl.when(i == 0)
    def _():
        acc_ref[...] = part

    @pl.when(i > 0)
    def _():
        acc_ref[...] += part

    @pl.when(i == (NPAD // blk) - 1)
    def _():
        o_ref[...] = jnp.dot(acc_ref[...] * (1.0 / N), fcw_ref[...],
                             preferred_element_type=jnp.float32) + fcb_ref[...]


_SC_MESH = plsc.VectorSubcoreMesh(core_axis_name="c", subcore_axis_name="s")
_Z16 = functools.partial(jnp.zeros, (16,), jnp.float32)


@functools.partial(
    pl.kernel,
    out_type=jax.ShapeDtypeStruct((2, NPAD, DH), jnp.float32),
    mesh=_SC_MESH,
    scratch_types=[
        pltpu.VMEM((GRP,), jnp.int32),
        pltpu.VMEM((GRP,), jnp.int32),
        pltpu.VMEM((GRP,), jnp.int32),
        pltpu.VMEM((CH,), jnp.int32),
        pltpu.VMEM((CH,), jnp.int32),
        pltpu.VMEM((CH,), jnp.int32),
        pltpu.VMEM((CH,), jnp.int32),
        pltpu.VMEM((CH,), jnp.int32),
        pltpu.VMEM((CH,), jnp.int32),
        pltpu.VMEM((CH,), jnp.float32),
        pltpu.VMEM((CH, DH), jnp.float32),
        pltpu.VMEM((CH, DH), jnp.float32),
        pltpu.VMEM((CH,), jnp.float32),
        pltpu.VMEM((CH,), jnp.float32),
        pltpu.VMEM((CNTP // NS,), jnp.float32),
        pltpu.VMEM_SHARED((NPAD, DH), jnp.float32),
        pltpu.VMEM_SHARED((CNTP,), jnp.float32),
        pltpu.SemaphoreType.DMA,
        pltpu.SemaphoreType.DMA,
        pltpu.SemaphoreType.DMA,
        pltpu.SemaphoreType.DMA,
    ],
)
def _rgcn_sc(src_hbm, dsts_hbm, et_hbm, xw_hbm, out_hbm,
             gs, gd, ge, segb0, segb1, gidx0, gidx1, dstb0, dstb1,
             onesb, rows0, rows1, invw0, invw1, zb1,
             agg_sp, cnt_sp, sr0, sr1, si0, si1):
    c = lax.axis_index("c")
    s = lax.axis_index("s")
    rowsb = (rows0, rows1)
    segb = (segb0, segb1)
    gidxb = (gidx0, gidx1)
    dstb = (dstb0, dstb1)
    invwb = (invw0, invw1)
    srs = (sr0, sr1)
    sis = (si0, si1)

    def zrow(e, _):
        for k in range(8):
            rows0[e, pl.ds(k * 16, 16)] = _Z16()
        return 0
    lax.fori_loop(0, CH, zrow, 0)

    def zzb(i, _):
        zb1[pl.ds(i * 16, 16)] = _Z16()
        return 0
    lax.fori_loop(0, (CNTP // NS) // 16, zzb, 0)

    for j in range(NPAD // NS // CH):
        pltpu.sync_copy(rows0, agg_sp.at[pl.ds(s * (NPAD // NS) + j * CH, CH)])
    pltpu.sync_copy(zb1, cnt_sp.at[pl.ds(s * (CNTP // NS), CNTP // NS)])
    for k in range(8):
        onesb[pl.ds(k * 16, 16)] = _Z16() + 1.0
    plsc.subcore_barrier()

    def cnt_grp(gi, _):
        gb = s * (EPAD // NS) + gi * GRP
        pltpu.sync_copy(dsts_hbm.at[pl.ds(gb, GRP)], gd)
        pltpu.sync_copy(et_hbm.at[pl.ds(gb, GRP)], ge)

        def cchunk(j, _):
            for k in range(8):
                sl = pl.ds(k * 16, 16)
                asl = pl.ds(j * CH + k * 16, 16)
                segb0[sl] = gd[asl] * RELS + ge[asl]
            pltpu.sync_copy(onesb, cnt_sp.at[segb0], add=True)
            return 0
        lax.fori_loop(0, GRP // CH, cchunk, 0)
        return 0
    lax.fori_loop(0, EPAD // NS // GRP, cnt_grp, 0)
    plsc.subcore_barrier()

    pltpu.sync_copy(cnt_sp.at[pl.ds(s * (CNTP // NS), CNTP // NS)], zb1)

    def inv_loop(i, _):
        sl = pl.ds(i * 16, 16)
        zb1[sl] = 1.0 / jnp.maximum(zb1[sl], 1.0)
        return 0
    lax.fori_loop(0, (CNTP // NS) // 16, inv_loop, 0)
    pltpu.sync_copy(zb1, cnt_sp.at[pl.ds(s * (CNTP // NS), CNTP // NS)])
    plsc.subcore_barrier()

    def issue(j, b):
        for k in range(8):
            sl = pl.ds(k * 16, 16)
            asl = pl.ds(j * CH + k * 16, 16)
            gidxb[b][sl] = ge[asl] * N + gs[asl]
            segb[b][sl] = gd[asl] * RELS + ge[asl]
            dstb[b][sl] = gd[asl]
        pltpu.async_copy(xw_hbm.at[gidxb[b]], rowsb[b], srs[b])
        pltpu.async_copy(cnt_sp.at[segb[b]], invwb[b], sis[b])

    def waitb(b):
        pltpu.make_async_copy(xw_hbm.at[pl.ds(0, CH)], rowsb[b],
                              srs[b]).wait()
        pltpu.make_async_copy(xw_hbm.at[0, pl.ds(0, CH)], invwb[b],
                              sis[b]).wait()

    def compute(b):
        rows = rowsb[b]
        invw = invwb[b]

        def escale(g, _):
            iw16 = invw[pl.ds(g * 16, 16)]
            for e2 in range(16):
                w_e = iw16[e2]
                row = g * 16 + e2
                for k in range(8):
                    sl = pl.ds(k * 16, 16)
                    rows[row, sl] = rows[row, sl] * w_e
            return 0
        lax.fori_loop(0, CH // 16, escale, 0)
        pltpu.sync_copy(rows, agg_sp.at[dstb[b]], add=True)

    def pair(p, _):
        issue(2 * p + 1, 1)
        waitb(0)
        compute(0)

        @pl.when(p < (GRP // CH) // 2 - 1)
        def _():
            issue(2 * p + 2, 0)
        waitb(1)
        compute(1)
        return 0

    def agg_grp(gi, _):
        gb = (c * NS + s) * (EPAD // 32) + gi * GRP
        pltpu.sync_copy(src_hbm.at[pl.ds(gb, GRP)], gs)
        pltpu.sync_copy(dsts_hbm.at[pl.ds(gb, GRP)], gd)
        pltpu.sync_copy(et_hbm.at[pl.ds(gb, GRP)], ge)
        issue(0, 0)
        lax.fori_loop(0, (GRP // CH) // 2, pair, 0)
        return 0
    lax.fori_loop(0, EPAD // 32 // GRP, agg_grp, 0)
    plsc.subcore_barrier()

    for j in range(NPAD // NS // CH):
        r0 = s * (NPAD // NS) + j * CH
        pltpu.sync_copy(agg_sp.at[pl.ds(r0, CH)], rows0)
        pltpu.sync_copy(rows0, out_hbm.at[c, pl.ds(r0, CH)])


@functools.partial(
    pl.kernel,
    out_type=(jax.ShapeDtypeStruct((2, NPAD, DH), jnp.float32),
              jax.ShapeDtypeStruct((2, 2, NPAD), jnp.float32),
              jax.ShapeDtypeStruct((2, 2, EPAD), jnp.float32)),
    mesh=_SC_MESH,
    scratch_types=[
        pltpu.VMEM((GRP,), jnp.int32),
        pltpu.VMEM((GRP,), jnp.int32),
        pltpu.VMEM((GRP,), jnp.float32),
        pltpu.VMEM((GRP,), jnp.float32),
        pltpu.VMEM((CHA, DH), jnp.float32),
        pltpu.VMEM((CHA, DH), jnp.float32),
        pltpu.VMEM((CHA, DH), jnp.float32),
        pltpu.VMEM((CHA, DH), jnp.float32),
        pltpu.VMEM((CHA,), jnp.float32),
        pltpu.VMEM((CHA,), jnp.float32),
        pltpu.VMEM((CHA,), jnp.float32),
        pltpu.VMEM((CHA,), jnp.float32),
        pltpu.VMEM((CHA,), jnp.int32),
        pltpu.VMEM((CHA,), jnp.int32),
        pltpu.VMEM((16,), jnp.float32),
        pltpu.VMEM((NS, 16), jnp.float32),
        pltpu.VMEM((NPAD // NS,), jnp.float32),
        pltpu.VMEM((16, 32), jnp.float32),
        pltpu.VMEM((16, 32), jnp.float32),
        pltpu.VMEM_SHARED((NPAD, DH), jnp.float32),
        pltpu.VMEM_SHARED((NPAD,), jnp.float32),
        pltpu.VMEM_SHARED((NPAD,), jnp.float32),
        pltpu.VMEM_SHARED((NS, 16), jnp.float32),
        pltpu.SemaphoreType.DMA,
        pltpu.SemaphoreType.DMA,
        pltpu.SemaphoreType.DMA,
        pltpu.SemaphoreType.DMA,
    ],
)
def _attn_sc(src_hbm, dstg_hbm, dsts_hbm,
             q01, q23, k01, k23, v01, v23,
             num_out, asum_out, alpha_out,
             g0, g1, albuf0, albuf1, qr0, qr1, kr0, kr1, aexp0, aexp1,
             aexp0x, aexp1x, dstsb, dstsbx, mbuf, mall, zb, pad0, pad1,
             num_sp, as0_sp, as1_sp, mx_sp, sq0, sq1, sk0, sk1):
    c = lax.axis_index("c")
    s = lax.axis_index("s")
    tbase = s * (EPAD // NS)
    qbufs = (qr0, qr1)
    kbufs = (kr0, kr1)
    sqs = (sq0, sq1)
    sks = (sk0, sk1)

    def lanesum(v, pad, slot):
        for sh in (8, 4, 2, 1):
            pad[slot, pl.ds(0, 16)] = v
            v = v + pad[slot, pl.ds(sh, 16)]
        return v[0]

    def zrow(e, _):
        for k in range(8):
            qr0[e, pl.ds(k * 16, 16)] = _Z16()
        return 0
    lax.fori_loop(0, CHA, zrow, 0)

    def zzb(i, _):
        zb[pl.ds(i * 16, 16)] = _Z16()
        return 0
    lax.fori_loop(0, (NPAD // NS) // 16, zzb, 0)
    for r in range(16):
        pad0[r, pl.ds(0, 16)] = _Z16()
        pad0[r, pl.ds(16, 16)] = _Z16()
        pad1[r, pl.ds(0, 16)] = _Z16()
        pad1[r, pl.ds(16, 16)] = _Z16()

    for j in range(NPAD // NS // CHA):
        pltpu.sync_copy(qr0, num_sp.at[pl.ds(s * (NPAD // NS) + j * CHA, CHA)])
    pltpu.sync_copy(zb, as0_sp.at[pl.ds(s * (NPAD // NS), NPAD // NS)])
    pltpu.sync_copy(zb, as1_sp.at[pl.ds(s * (NPAD // NS), NPAD // NS)])
    plsc.subcore_barrier()

    def phase_a(qt, kt):
        def issue(jj, b):
            pltpu.async_copy(qt.at[g1.at[pl.ds(jj * CHA, CHA)]],
                             qbufs[b], sqs[b])
            pltpu.async_copy(kt.at[g0.at[pl.ds(jj * CHA, CHA)]],
                             kbufs[b], sks[b])

        def waitb(b):
            pltpu.make_async_copy(qt.at[pl.ds(0, CHA)], qbufs[b],
                                  sqs[b]).wait()
            pltpu.make_async_copy(kt.at[pl.ds(0, CHA)], kbufs[b],
                                  sks[b]).wait()

        lane = lax.iota(jnp.int32, 16)

        def compute(jj, b, m):
            qb = qbufs[b]
            kb = kbufs[b]

            def egrp(g, m):
                z0 = _Z16()
                z1 = _Z16()
                for e2 in range(16):
                    row = g * 16 + e2
                    p0 = qb[row, pl.ds(0, 16)] * kb[row, pl.ds(0, 16)]
                    for k in range(1, 4):
                        sl = pl.ds(k * 16, 16)
                        p0 = p0 + qb[row, sl] * kb[row, sl]
                    p1 = qb[row, pl.ds(64, 16)] * kb[row, pl.ds(64, 16)]
                    for k in range(5, 8):
                        sl = pl.ds(k * 16, 16)
                        p1 = p1 + qb[row, sl] * kb[row, sl]
                    s0 = lanesum(p0, pad0, e2)
                    s1 = lanesum(p1, pad1, e2)
                    z0 = jnp.where(lane == e2, s0, z0)
                    z1 = jnp.where(lane == e2, s1, z1)
                albuf0[pl.ds(jj * CHA + g * 16, 16)] = z0
                albuf1[pl.ds(jj * CHA + g * 16, 16)] = z1
                return jnp.maximum(m, jnp.maximum(z0, z1))
            return lax.fori_loop(0, CHA // 16, egrp, m)

        def pair(p, m):
            issue(2 * p + 1, 1)
            waitb(0)
            m = compute(2 * p, 0, m)

            @pl.when(p < (GRP // CHA) // 2 - 1)
            def _():
                issue(2 * p + 2, 0)
            waitb(1)
            return compute(2 * p + 1, 1, m)

        def grp_loop(gi, m):
            gb = tbase + gi * GRP
            pltpu.sync_copy(src_hbm.at[pl.ds(gb, GRP)], g0)
            pltpu.sync_copy(dstg_hbm.at[pl.ds(gb, GRP)], g1)
            issue(0, 0)
            m = lax.fori_loop(0, (GRP // CHA) // 2, pair, m)
            pltpu.sync_copy(albuf0, alpha_out.at[c, 0, pl.ds(gb, GRP)])
            pltpu.sync_copy(albuf1, alpha_out.at[c, 1, pl.ds(gb, GRP)])
            return m
        m = lax.fori_loop(0, EPAD // NS // GRP, grp_loop,
                          jnp.full((16,), -3.0e38, jnp.float32))
        mbuf[pl.ds(0, 16)] = m

    @pl.when(c == 0)
    def _():
        phase_a(q01, k01)

    @pl.when(c == 1)
    def _():
        phase_a(q23, k23)

    pltpu.sync_copy(mbuf, mx_sp.at[s])
    plsc.subcore_barrier()
    pltpu.sync_copy(mx_sp, mall)
    gv = mall[0, pl.ds(0, 16)]
    for t in range(1, NS):
        gv = jnp.maximum(gv, mall[t, pl.ds(0, 16)])
    gmax = gv[0]
    for t in range(1, 16):
        gmax = jnp.maximum(gmax, gv[t])

    def phase_b(vt):
        ae0 = (aexp0, aexp0x)
        ae1 = (aexp1, aexp1x)
        dsb = (dstsb, dstsbx)

        def issuev(jj, b):
            pltpu.async_copy(vt.at[g0.at[pl.ds(jj * CHA, CHA)]],
                             qbufs[b], sqs[b])

        def waitv(b):
            pltpu.make_async_copy(vt.at[pl.ds(0, CHA)], qbufs[b],
                                  sqs[b]).wait()

        def waitsc(b):
            pltpu.make_async_copy(ae0[b], as0_sp.at[dsb[b]], sks[b]).wait()
            pltpu.make_async_copy(ae1[b], as1_sp.at[dsb[b]], sks[b]).wait()
            pltpu.make_async_copy(kbufs[b], num_sp.at[dsb[b]], sks[b]).wait()

        def computeb(jj, b):
            vb = qbufs[b]
            sb = kbufs[b]
            for k in range(CHA // 16):
                sl = pl.ds(k * 16, 16)
                asl = pl.ds(jj * CHA + k * 16, 16)
                ae0[b][sl] = jnp.exp(albuf0[asl] - gmax)
                ae1[b][sl] = jnp.exp(albuf1[asl] - gmax)
                dsb[b][sl] = g1[asl]

            def escale(g, _):
                a0v = ae0[b][pl.ds(g * 16, 16)]
                a1v = ae1[b][pl.ds(g * 16, 16)]
                for e2 in range(16):
                    row = g * 16 + e2
                    a0 = a0v[e2]
                    a1 = a1v[e2]
                    for k in range(4):
                        sl = pl.ds(k * 16, 16)
                        sb[row, sl] = vb[row, sl] * a0
                    for k in range(4, 8):
                        sl = pl.ds(k * 16, 16)
                        sb[row, sl] = vb[row, sl] * a1
                return 0
            lax.fori_loop(0, CHA // 16, escale, 0)
            pltpu.async_copy(ae0[b], as0_sp.at[dsb[b]], sks[b], add=True)
            pltpu.async_copy(ae1[b], as1_sp.at[dsb[b]], sks[b], add=True)
            pltpu.async_copy(sb, num_sp.at[dsb[b]], sks[b], add=True)

        def pairb(p, _):
            issuev(2 * p + 1, 1)
            waitv(0)

            @pl.when(p > 0)
            def _():
                waitsc(0)
            computeb(2 * p, 0)

            @pl.when(p < (GRP // CHA) // 2 - 1)
            def _():
                issuev(2 * p + 2, 0)
            waitv(1)

            @pl.when(p > 0)
            def _():
                waitsc(1)
            computeb(2 * p + 1, 1)
            return 0

        def grpb(gi, _):
            gb = tbase + gi * GRP
            pltpu.sync_copy(src_hbm.at[pl.ds(gb, GRP)], g0)
            pltpu.sync_copy(dsts_hbm.at[pl.ds(gb, GRP)], g1)
            pltpu.sync_copy(alpha_out.at[c, 0, pl.ds(gb, GRP)], albuf0)
            pltpu.sync_copy(alpha_out.at[c, 1, pl.ds(gb, GRP)], albuf1)
            issuev(0, 0)
            lax.fori_loop(0, (GRP // CHA) // 2, pairb, 0)
            waitsc(0)
            waitsc(1)
            return 0
        lax.fori_loop(0, EPAD // NS // GRP, grpb, 0)

    @pl.when(c == 0)
    def _():
        phase_b(v01)

    @pl.when(c == 1)
    def _():
        phase_b(v23)

    plsc.subcore_barrier()

    for j in range(NPAD // NS // CHA):
        r0 = s * (NPAD // NS) + j * CHA
        pltpu.sync_copy(num_sp.at[pl.ds(r0, CHA)], kr0)
        pltpu.sync_copy(kr0, num_out.at[c, pl.ds(r0, CHA)])
    a0 = s * (NPAD // NS)
    pltpu.sync_copy(as0_sp.at[pl.ds(a0, NPAD // NS)], zb)
    pltpu.sync_copy(zb, asum_out.at[c, 0, pl.ds(a0, NPAD // NS)])
    pltpu.sync_copy(as1_sp.at[pl.ds(a0, NPAD // NS)], zb)
    pltpu.sync_copy(zb, asum_out.at[c, 1, pl.ds(a0, NPAD // NS)])


def kernel(x, edge_index, edge_type, rgcn_w, rgcn_root, rgcn_bias,
           wq, bq, wk, bk, wv, bv, wskip, bskip, fc_w, fc_b):
    src = edge_index[0].astype(jnp.int32)
    dst = edge_index[1].astype(jnp.int32)
    et = edge_type.astype(jnp.int32)

    pcnt = EPAD - E
    pidx = jnp.arange(pcnt, dtype=jnp.int32)
    src_g = jnp.concatenate([src, (pidx * 997) % N])
    dst_g = jnp.concatenate([dst, (pidx * 1013 + 7) % N])
    dst_s = jnp.concatenate([dst, N + (pidx % 16)])
    et_g = jnp.concatenate([et, jnp.zeros((pcnt,), jnp.int32)])

    w7 = jnp.concatenate([rgcn_w, rgcn_root[None]], axis=0)
    xw7 = pl.pallas_call(
        _mm7_body,
        grid=(N // NB, 7),
        in_specs=[pl.BlockSpec((NB, 768), lambda i, r: (i, 0)),
                  pl.BlockSpec((1, 768, DH), lambda i, r: (r, 0, 0))],
        out_specs=pl.BlockSpec((1, NB, DH), lambda i, r: (r, i, 0)),
        out_shape=jax.ShapeDtypeStruct((7, N, DH), jnp.float32),
    )(x, w7)
    xw_flat = xw7[:RELS].reshape(RELS * N, DH)
    xroot = xw7[RELS]

    agg2 = _rgcn_sc(src_g, dst_s, et_g, xw_flat)

    wcat = jnp.concatenate([wq * 0.125, wk, wv, wskip], axis=1)
    bcat = jnp.concatenate([bq * 0.125, bk, bv, bskip])[None]
    outs = pl.pallas_call(
        _qkv_body,
        grid=(N // NB,),
        in_specs=[pl.BlockSpec((2, NB, DH), lambda i: (0, i, 0)),
                  pl.BlockSpec((NB, DH), lambda i: (i, 0)),
                  pl.BlockSpec((1, DH), lambda i: (0, 0)),
                  pl.BlockSpec((DH, 1024), lambda i: (0, 0)),
                  pl.BlockSpec((1, 1024), lambda i: (0, 0))],
        out_specs=[pl.BlockSpec((NB, DH), lambda i: (i, 0))] * 6
        + [pl.BlockSpec((NB, 256), lambda i: (i, 0))],
        out_shape=[jax.ShapeDtypeStruct((N, DH), jnp.float32)] * 6
        + [jax.ShapeDtypeStruct((N, 256), jnp.float32)],
    )(agg2, xroot, rgcn_bias[None], wcat, bcat)
    q01, q23, k01, k23, v01, v23, skip = outs

    num2, asum2, _ = _attn_sc(src_g, dst_g, dst_s, q01, q23, k01, k23,
                              v01, v23)

    skip_p = jnp.pad(skip, ((0, NPAD - N), (0, 0)))
    fcw_p = jnp.pad(fc_w, ((0, 0), (0, 125)))
    fcb_p = jnp.pad(fc_b, (0, 125))[None]
    out = pl.pallas_call(
        _final_body,
        grid=(NPAD // 1024,),
        in_specs=[pl.BlockSpec((2, 1024, DH), lambda i: (0, i, 0)),
                  pl.BlockSpec((2, 2, 1024), lambda i: (0, 0, i)),
                  pl.BlockSpec((1024, 256), lambda i: (i, 0)),
                  pl.BlockSpec((1, 256), lambda i: (0, 0)),
                  pl.BlockSpec((256, 128), lambda i: (0, 0)),
                  pl.BlockSpec((1, 128), lambda i: (0, 0))],
        out_specs=pl.BlockSpec((1, 128), lambda i: (0, 0)),
        out_shape=jax.ShapeDtypeStruct((1, 128), jnp.float32),
        scratch_shapes=[pltpu.VMEM((1, 256), jnp.float32)],
    )(num2, asum2, skip_p, bskip[None], fcw_p, fcb_p)
    return out[:, :3]


# speedup vs baseline: 26.2111x; 1.0050x over previous
import functools

import jax
import jax.numpy as jnp
from jax import lax
from jax.experimental import pallas as pl
from jax.experimental.pallas import tpu as pltpu
from jax.experimental.pallas import tpu_sc as plsc

N = 10000
E = 160000
RELS = 6
DH = 128
NPAD = 10240
EPAD = 163840
CH = 128
CNTP = 60416
NS = 16
NB = 1000
CHA = 64
GRP = 1024


def _mm7_body(x_ref, w_ref, o_ref):
    o_ref[0] = jnp.dot(x_ref[...], w_ref[0], preferred_element_type=jnp.float32)


def _qkv_body(agg_ref, xroot_ref, bias_ref, wcat_ref, bcat_ref,
              q01_ref, q23_ref, k01_ref, k23_ref, v01_ref, v23_ref, skip_ref):
    h = agg_ref[0] + agg_ref[1] + xroot_ref[...] + bias_ref[...]
    h = jnp.maximum(h, 0.0)
    y = jnp.dot(h, wcat_ref[...], preferred_element_type=jnp.float32) + bcat_ref[...]
    q01_ref[...] = y[:, 0:128]
    q23_ref[...] = y[:, 128:256]
    k01_ref[...] = y[:, 256:384]
    k23_ref[...] = y[:, 384:512]
    v01_ref[...] = y[:, 512:640]
    v23_ref[...] = y[:, 640:768]
    skip_ref[...] = y[:, 768:1024]


def _final_body(num_ref, asum_ref, skip_ref, bskip_ref, fcw_ref, fcb_ref,
                o_ref, acc_ref):
    i = pl.program_id(0)
    blk = 1024
    a = asum_ref[...]
    a00 = jnp.maximum(a[0, 0, :], 1e-30).reshape(blk, 1)
    a01 = jnp.maximum(a[0, 1, :], 1e-30).reshape(blk, 1)
    a10 = jnp.maximum(a[1, 0, :], 1e-30).reshape(blk, 1)
    a11 = jnp.maximum(a[1, 1, :], 1e-30).reshape(blk, 1)
    nm = num_ref[...]
    o = jnp.concatenate(
        [nm[0, :, 0:64] / a00, nm[0, :, 64:128] / a01,
         nm[1, :, 0:64] / a10, nm[1, :, 64:128] / a11], axis=1)
    o = jnp.maximum(o + skip_ref[...] + bskip_ref[...], 0.0)
    rows = i * blk + lax.broadcasted_iota(jnp.int32, (blk, 1), 0)
    o = jnp.where(rows < N, o, 0.0)
    part = jnp.sum(o, axis=0, keepdims=True)

    @pl.when(i == 0)
    def _():
        acc_ref[...] = part

    @pl.when(i > 0)
    def _():
        acc_ref[...] += part

    @pl.when(i == (NPAD // blk) - 1)
    def _():
        o_ref[...] = jnp.dot(acc_ref[...] * (1.0 / N), fcw_ref[...],
                             preferred_element_type=jnp.float32) + fcb_ref[...]


_SC_MESH = plsc.VectorSubcoreMesh(core_axis_name="c", subcore_axis_name="s")
_Z16 = functools.partial(jnp.zeros, (16,), jnp.float32)


@functools.partial(
    pl.kernel,
    out_type=jax.ShapeDtypeStruct((2, NPAD, DH), jnp.float32),
    mesh=_SC_MESH,
    scratch_types=[
        pltpu.VMEM((GRP,), jnp.int32),
        pltpu.VMEM((GRP,), jnp.int32),
        pltpu.VMEM((GRP,), jnp.int32),
        pltpu.VMEM((CH,), jnp.int32),
        pltpu.VMEM((CH,), jnp.int32),
        pltpu.VMEM((CH,), jnp.int32),
        pltpu.VMEM((CH,), jnp.int32),
        pltpu.VMEM((CH,), jnp.int32),
        pltpu.VMEM((CH,), jnp.int32),
        pltpu.VMEM((CH,), jnp.float32),
        pltpu.VMEM((CH, DH), jnp.float32),
        pltpu.VMEM((CH, DH), jnp.float32),
        pltpu.VMEM((CH,), jnp.float32),
        pltpu.VMEM((CH,), jnp.float32),
        pltpu.VMEM((CNTP // NS,), jnp.float32),
        pltpu.VMEM_SHARED((NPAD, DH), jnp.float32),
        pltpu.VMEM_SHARED((CNTP,), jnp.float32),
        pltpu.SemaphoreType.DMA,
        pltpu.SemaphoreType.DMA,
        pltpu.SemaphoreType.DMA,
        pltpu.SemaphoreType.DMA,
    ],
)
def _rgcn_sc(src_hbm, dsts_hbm, et_hbm, xw_hbm, out_hbm,
             gs, gd, ge, segb0, segb1, gidx0, gidx1, dstb0, dstb1,
             onesb, rows0, rows1, invw0, invw1, zb1,
             agg_sp, cnt_sp, sr0, sr1, si0, si1):
    c = lax.axis_index("c")
    s = lax.axis_index("s")
    rowsb = (rows0, rows1)
    segb = (segb0, segb1)
    gidxb = (gidx0, gidx1)
    dstb = (dstb0, dstb1)
    invwb = (invw0, invw1)
    srs = (sr0, sr1)
    sis = (si0, si1)

    def zrow(e, _):
        for k in range(8):
            rows0[e, pl.ds(k * 16, 16)] = _Z16()
        return 0
    lax.fori_loop(0, CH, zrow, 0)

    def zzb(i, _):
        zb1[pl.ds(i * 16, 16)] = _Z16()
        return 0
    lax.fori_loop(0, (CNTP // NS) // 16, zzb, 0)

    for j in range(NPAD // NS // CH):
        pltpu.sync_copy(rows0, agg_sp.at[pl.ds(s * (NPAD // NS) + j * CH, CH)])
    pltpu.sync_copy(zb1, cnt_sp.at[pl.ds(s * (CNTP // NS), CNTP // NS)])
    for k in range(8):
        onesb[pl.ds(k * 16, 16)] = _Z16() + 1.0
    plsc.subcore_barrier()

    def cnt_grp(gi, _):
        gb = s * (EPAD // NS) + gi * GRP
        pltpu.sync_copy(dsts_hbm.at[pl.ds(gb, GRP)], gd)
        pltpu.sync_copy(et_hbm.at[pl.ds(gb, GRP)], ge)

        def cpair(p, _):
            @pl.when(p > 0)
            def _():
                pltpu.make_async_copy(onesb, cnt_sp.at[segb0], sr0).wait()
                pltpu.make_async_copy(onesb, cnt_sp.at[segb1], sr1).wait()
            for k in range(8):
                sl = pl.ds(k * 16, 16)
                asl = pl.ds(2 * p * CH + k * 16, 16)
                segb0[sl] = gd[asl] * RELS + ge[asl]
            pltpu.async_copy(onesb, cnt_sp.at[segb0], sr0, add=True)
            for k in range(8):
                sl = pl.ds(k * 16, 16)
                asl = pl.ds((2 * p + 1) * CH + k * 16, 16)
                segb1[sl] = gd[asl] * RELS + ge[asl]
            pltpu.async_copy(onesb, cnt_sp.at[segb1], sr1, add=True)
            return 0
        lax.fori_loop(0, GRP // CH // 2, cpair, 0)
        pltpu.make_async_copy(onesb, cnt_sp.at[segb0], sr0).wait()
        pltpu.make_async_copy(onesb, cnt_sp.at[segb1], sr1).wait()
        return 0
    lax.fori_loop(0, EPAD // NS // GRP, cnt_grp, 0)
    plsc.subcore_barrier()

    pltpu.sync_copy(cnt_sp.at[pl.ds(s * (CNTP // NS), CNTP // NS)], zb1)

    def inv_loop(i, _):
        sl = pl.ds(i * 16, 16)
        zb1[sl] = 1.0 / jnp.maximum(zb1[sl], 1.0)
        return 0
    lax.fori_loop(0, (CNTP // NS) // 16, inv_loop, 0)
    pltpu.sync_copy(zb1, cnt_sp.at[pl.ds(s * (CNTP // NS), CNTP // NS)])
    plsc.subcore_barrier()

    def issue(j, b):
        for k in range(8):
            sl = pl.ds(k * 16, 16)
            asl = pl.ds(j * CH + k * 16, 16)
            gidxb[b][sl] = ge[asl] * N + gs[asl]
            segb[b][sl] = gd[asl] * RELS + ge[asl]
            dstb[b][sl] = gd[asl]
        pltpu.async_copy(xw_hbm.at[gidxb[b]], rowsb[b], srs[b])
        pltpu.async_copy(cnt_sp.at[segb[b]], invwb[b], sis[b])

    def waitb(b):
        pltpu.make_async_copy(xw_hbm.at[pl.ds(0, CH)], rowsb[b],
                              srs[b]).wait()
        pltpu.make_async_copy(xw_hbm.at[0, pl.ds(0, CH)], invwb[b],
                              sis[b]).wait()

    def compute(b):
        rows = rowsb[b]
        invw = invwb[b]

        def escale(g, _):
            iw16 = invw[pl.ds(g * 16, 16)]
            for e2 in range(16):
                w_e = iw16[e2]
                row = g * 16 + e2
                for k in range(8):
                    sl = pl.ds(k * 16, 16)
                    rows[row, sl] = rows[row, sl] * w_e
            return 0
        lax.fori_loop(0, CH // 16, escale, 0)
        pltpu.sync_copy(rows, agg_sp.at[dstb[b]], add=True)

    def pair(p, _):
        issue(2 * p + 1, 1)
        waitb(0)
        compute(0)

        @pl.when(p < (GRP // CH) // 2 - 1)
        def _():
            issue(2 * p + 2, 0)
        waitb(1)
        compute(1)
        return 0

    def agg_grp(gi, _):
        gb = (c * NS + s) * (EPAD // 32) + gi * GRP
        pltpu.sync_copy(src_hbm.at[pl.ds(gb, GRP)], gs)
        pltpu.sync_copy(dsts_hbm.at[pl.ds(gb, GRP)], gd)
        pltpu.sync_copy(et_hbm.at[pl.ds(gb, GRP)], ge)
        issue(0, 0)
        lax.fori_loop(0, (GRP // CH) // 2, pair, 0)
        return 0
    lax.fori_loop(0, EPAD // 32 // GRP, agg_grp, 0)
    plsc.subcore_barrier()

    for j in range(NPAD // NS // CH):
        r0 = s * (NPAD // NS) + j * CH
        pltpu.sync_copy(agg_sp.at[pl.ds(r0, CH)], rows0)
        pltpu.sync_copy(rows0, out_hbm.at[c, pl.ds(r0, CH)])


@functools.partial(
    pl.kernel,
    out_type=(jax.ShapeDtypeStruct((2, NPAD, DH), jnp.float32),
              jax.ShapeDtypeStruct((2, 2, NPAD), jnp.float32),
              jax.ShapeDtypeStruct((2, 2, EPAD), jnp.float32)),
    mesh=_SC_MESH,
    scratch_types=[
        pltpu.VMEM((GRP,), jnp.int32),
        pltpu.VMEM((GRP,), jnp.int32),
        pltpu.VMEM((GRP,), jnp.float32),
        pltpu.VMEM((GRP,), jnp.float32),
        pltpu.VMEM((CHA, DH), jnp.float32),
        pltpu.VMEM((CHA, DH), jnp.float32),
        pltpu.VMEM((CHA, DH), jnp.float32),
        pltpu.VMEM((CHA, DH), jnp.float32),
        pltpu.VMEM((CHA,), jnp.float32),
        pltpu.VMEM((CHA,), jnp.float32),
        pltpu.VMEM((CHA,), jnp.float32),
        pltpu.VMEM((CHA,), jnp.float32),
        pltpu.VMEM((CHA,), jnp.int32),
        pltpu.VMEM((CHA,), jnp.int32),
        pltpu.VMEM((16,), jnp.float32),
        pltpu.VMEM((NS, 16), jnp.float32),
        pltpu.VMEM((NPAD // NS,), jnp.float32),
        pltpu.VMEM((16, 32), jnp.float32),
        pltpu.VMEM((16, 32), jnp.float32),
        pltpu.VMEM_SHARED((NPAD, DH), jnp.float32),
        pltpu.VMEM_SHARED((NPAD,), jnp.float32),
        pltpu.VMEM_SHARED((NPAD,), jnp.float32),
        pltpu.VMEM_SHARED((NS, 16), jnp.float32),
        pltpu.SemaphoreType.DMA,
        pltpu.SemaphoreType.DMA,
        pltpu.SemaphoreType.DMA,
        pltpu.SemaphoreType.DMA,
    ],
)
def _attn_sc(src_hbm, dstg_hbm, dsts_hbm,
             q01, q23, k01, k23, v01, v23,
             num_out, asum_out, alpha_out,
             g0, g1, albuf0, albuf1, qr0, qr1, kr0, kr1, aexp0, aexp1,
             aexp0x, aexp1x, dstsb, dstsbx, mbuf, mall, zb, pad0, pad1,
             num_sp, as0_sp, as1_sp, mx_sp, sq0, sq1, sk0, sk1):
    c = lax.axis_index("c")
    s = lax.axis_index("s")
    tbase = s * (EPAD // NS)
    qbufs = (qr0, qr1)
    kbufs = (kr0, kr1)
    sqs = (sq0, sq1)
    sks = (sk0, sk1)

    def lanesum(v, pad, slot):
        for sh in (8, 4, 2, 1):
            pad[slot, pl.ds(0, 16)] = v
            v = v + pad[slot, pl.ds(sh, 16)]
        return v[0]

    def zrow(e, _):
        for k in range(8):
            qr0[e, pl.ds(k * 16, 16)] = _Z16()
        return 0
    lax.fori_loop(0, CHA, zrow, 0)

    def zzb(i, _):
        zb[pl.ds(i * 16, 16)] = _Z16()
        return 0
    lax.fori_loop(0, (NPAD // NS) // 16, zzb, 0)
    for r in range(16):
        pad0[r, pl.ds(0, 16)] = _Z16()
        pad0[r, pl.ds(16, 16)] = _Z16()
        pad1[r, pl.ds(0, 16)] = _Z16()
        pad1[r, pl.ds(16, 16)] = _Z16()

    for j in range(NPAD // NS // CHA):
        pltpu.sync_copy(qr0, num_sp.at[pl.ds(s * (NPAD // NS) + j * CHA, CHA)])
    pltpu.sync_copy(zb, as0_sp.at[pl.ds(s * (NPAD // NS), NPAD // NS)])
    pltpu.sync_copy(zb, as1_sp.at[pl.ds(s * (NPAD // NS), NPAD // NS)])
    plsc.subcore_barrier()

    def phase_a(qt, kt):
        def issue(jj, b):
            pltpu.async_copy(qt.at[g1.at[pl.ds(jj * CHA, CHA)]],
                             qbufs[b], sqs[b])
            pltpu.async_copy(kt.at[g0.at[pl.ds(jj * CHA, CHA)]],
                             kbufs[b], sks[b])

        def waitb(b):
            pltpu.make_async_copy(qt.at[pl.ds(0, CHA)], qbufs[b],
                                  sqs[b]).wait()
            pltpu.make_async_copy(kt.at[pl.ds(0, CHA)], kbufs[b],
                                  sks[b]).wait()

        lane = lax.iota(jnp.int32, 16)

        def compute(jj, b, m):
            qb = qbufs[b]
            kb = kbufs[b]

            def egrp(g, m):
                z0 = _Z16()
                z1 = _Z16()
                for e2 in range(16):
                    row = g * 16 + e2
                    p0 = qb[row, pl.ds(0, 16)] * kb[row, pl.ds(0, 16)]
                    for k in range(1, 4):
                        sl = pl.ds(k * 16, 16)
                        p0 = p0 + qb[row, sl] * kb[row, sl]
                    p1 = qb[row, pl.ds(64, 16)] * kb[row, pl.ds(64, 16)]
                    for k in range(5, 8):
                        sl = pl.ds(k * 16, 16)
                        p1 = p1 + qb[row, sl] * kb[row, sl]
                    s0 = lanesum(p0, pad0, e2)
                    s1 = lanesum(p1, pad1, e2)
                    z0 = jnp.where(lane == e2, s0, z0)
                    z1 = jnp.where(lane == e2, s1, z1)
                albuf0[pl.ds(jj * CHA + g * 16, 16)] = z0
                albuf1[pl.ds(jj * CHA + g * 16, 16)] = z1
                return jnp.maximum(m, jnp.maximum(z0, z1))
            return lax.fori_loop(0, CHA // 16, egrp, m)

        def pair(p, m):
            issue(2 * p + 1, 1)
            waitb(0)
            m = compute(2 * p, 0, m)

            @pl.when(p < (GRP // CHA) // 2 - 1)
            def _():
                issue(2 * p + 2, 0)
            waitb(1)
            return compute(2 * p + 1, 1, m)

        def grp_loop(gi, m):
            gb = tbase + gi * GRP
            pltpu.sync_copy(src_hbm.at[pl.ds(gb, GRP)], g0)
            pltpu.sync_copy(dstg_hbm.at[pl.ds(gb, GRP)], g1)
            issue(0, 0)
            m = lax.fori_loop(0, (GRP // CHA) // 2, pair, m)
            pltpu.sync_copy(albuf0, alpha_out.at[c, 0, pl.ds(gb, GRP)])
            pltpu.sync_copy(albuf1, alpha_out.at[c, 1, pl.ds(gb, GRP)])
            return m
        m = lax.fori_loop(0, EPAD // NS // GRP, grp_loop,
                          jnp.full((16,), -3.0e38, jnp.float32))
        mbuf[pl.ds(0, 16)] = m

    @pl.when(c == 0)
    def _():
        phase_a(q01, k01)

    @pl.when(c == 1)
    def _():
        phase_a(q23, k23)

    pltpu.sync_copy(mbuf, mx_sp.at[s])
    plsc.subcore_barrier()
    pltpu.sync_copy(mx_sp, mall)
    gv = mall[0, pl.ds(0, 16)]
    for t in range(1, NS):
        gv = jnp.maximum(gv, mall[t, pl.ds(0, 16)])
    gmax = gv[0]
    for t in range(1, 16):
        gmax = jnp.maximum(gmax, gv[t])

    def phase_b(vt):
        ae0 = (aexp0, aexp0x)
        ae1 = (aexp1, aexp1x)
        dsb = (dstsb, dstsbx)

        def issuev(jj, b):
            pltpu.async_copy(vt.at[g0.at[pl.ds(jj * CHA, CHA)]],
                             qbufs[b], sqs[b])

        def waitv(b):
            pltpu.make_async_copy(vt.at[pl.ds(0, CHA)], qbufs[b],
                                  sqs[b]).wait()

        def waitsc(b):
            pltpu.make_async_copy(ae0[b], as0_sp.at[dsb[b]], sks[b]).wait()
            pltpu.make_async_copy(ae1[b], as1_sp.at[dsb[b]], sks[b]).wait()
            pltpu.make_async_copy(kbufs[b], num_sp.at[dsb[b]], sks[b]).wait()

        def computeb(jj, b):
            vb = qbufs[b]
            sb = kbufs[b]
            for k in range(CHA // 16):
                sl = pl.ds(k * 16, 16)
                asl = pl.ds(jj * CHA + k * 16, 16)
                ae0[b][sl] = jnp.exp(albuf0[asl] - gmax)
                ae1[b][sl] = jnp.exp(albuf1[asl] - gmax)
                dsb[b][sl] = g1[asl]

            def escale(g, _):
                a0v = ae0[b][pl.ds(g * 16, 16)]
                a1v = ae1[b][pl.ds(g * 16, 16)]
                for e2 in range(16):
                    row = g * 16 + e2
                    a0 = a0v[e2]
                    a1 = a1v[e2]
                    for k in range(4):
                        sl = pl.ds(k * 16, 16)
                        sb[row, sl] = vb[row, sl] * a0
                    for k in range(4, 8):
                        sl = pl.ds(k * 16, 16)
                        sb[row, sl] = vb[row, sl] * a1
                return 0
            lax.fori_loop(0, CHA // 16, escale, 0)
            pltpu.async_copy(ae0[b], as0_sp.at[dsb[b]], sks[b], add=True)
            pltpu.async_copy(ae1[b], as1_sp.at[dsb[b]], sks[b], add=True)
            pltpu.async_copy(sb, num_sp.at[dsb[b]], sks[b], add=True)

        def pairb(p, _):
            issuev(2 * p + 1, 1)
            waitv(0)

            @pl.when(p > 0)
            def _():
                waitsc(0)
            computeb(2 * p, 0)

            @pl.when(p < (GRP // CHA) // 2 - 1)
            def _():
                issuev(2 * p + 2, 0)
            waitv(1)

            @pl.when(p > 0)
            def _():
                waitsc(1)
            computeb(2 * p + 1, 1)
            return 0

        def grpb(gi, _):
            gb = tbase + gi * GRP
            pltpu.sync_copy(src_hbm.at[pl.ds(gb, GRP)], g0)
            pltpu.sync_copy(dsts_hbm.at[pl.ds(gb, GRP)], g1)
            pltpu.sync_copy(alpha_out.at[c, 0, pl.ds(gb, GRP)], albuf0)
            pltpu.sync_copy(alpha_out.at[c, 1, pl.ds(gb, GRP)], albuf1)
            issuev(0, 0)
            lax.fori_loop(0, (GRP // CHA) // 2, pairb, 0)
            waitsc(0)
            waitsc(1)
            return 0
        lax.fori_loop(0, EPAD // NS // GRP, grpb, 0)

    @pl.when(c == 0)
    def _():
        phase_b(v01)

    @pl.when(c == 1)
    def _():
        phase_b(v23)

    plsc.subcore_barrier()

    for j in range(NPAD // NS // CHA):
        r0 = s * (NPAD // NS) + j * CHA
        pltpu.sync_copy(num_sp.at[pl.ds(r0, CHA)], kr0)
        pltpu.sync_copy(kr0, num_out.at[c, pl.ds(r0, CHA)])
    a0 = s * (NPAD // NS)
    pltpu.sync_copy(as0_sp.at[pl.ds(a0, NPAD // NS)], zb)
    pltpu.sync_copy(zb, asum_out.at[c, 0, pl.ds(a0, NPAD // NS)])
    pltpu.sync_copy(as1_sp.at[pl.ds(a0, NPAD // NS)], zb)
    pltpu.sync_copy(zb, asum_out.at[c, 1, pl.ds(a0, NPAD // NS)])


def kernel(x, edge_index, edge_type, rgcn_w, rgcn_root, rgcn_bias,
           wq, bq, wk, bk, wv, bv, wskip, bskip, fc_w, fc_b):
    src = edge_index[0].astype(jnp.int32)
    dst = edge_index[1].astype(jnp.int32)
    et = edge_type.astype(jnp.int32)

    pcnt = EPAD - E
    pidx = jnp.arange(pcnt, dtype=jnp.int32)
    src_g = jnp.concatenate([src, (pidx * 997) % N])
    dst_g = jnp.concatenate([dst, (pidx * 1013 + 7) % N])
    dst_s = jnp.concatenate([dst, N + (pidx % 16)])
    et_g = jnp.concatenate([et, jnp.zeros((pcnt,), jnp.int32)])

    w7 = jnp.concatenate([rgcn_w, rgcn_root[None]], axis=0)
    xw7 = pl.pallas_call(
        _mm7_body,
        grid=(N // NB, 7),
        in_specs=[pl.BlockSpec((NB, 768), lambda i, r: (i, 0)),
                  pl.BlockSpec((1, 768, DH), lambda i, r: (r, 0, 0))],
        out_specs=pl.BlockSpec((1, NB, DH), lambda i, r: (r, i, 0)),
        out_shape=jax.ShapeDtypeStruct((7, N, DH), jnp.float32),
    )(x, w7)
    xw_flat = xw7[:RELS].reshape(RELS * N, DH)
    xroot = xw7[RELS]

    agg2 = _rgcn_sc(src_g, dst_s, et_g, xw_flat)

    wcat = jnp.concatenate([wq * 0.125, wk, wv, wskip], axis=1)
    bcat = jnp.concatenate([bq * 0.125, bk, bv, bskip])[None]
    outs = pl.pallas_call(
        _qkv_body,
        grid=(N // NB,),
        in_specs=[pl.BlockSpec((2, NB, DH), lambda i: (0, i, 0)),
                  pl.BlockSpec((NB, DH), lambda i: (i, 0)),
                  pl.BlockSpec((1, DH), lambda i: (0, 0)),
                  pl.BlockSpec((DH, 1024), lambda i: (0, 0)),
                  pl.BlockSpec((1, 1024), lambda i: (0, 0))],
        out_specs=[pl.BlockSpec((NB, DH), lambda i: (i, 0))] * 6
        + [pl.BlockSpec((NB, 256), lambda i: (i, 0))],
        out_shape=[jax.ShapeDtypeStruct((N, DH), jnp.float32)] * 6
        + [jax.ShapeDtypeStruct((N, 256), jnp.float32)],
    )(agg2, xroot, rgcn_bias[None], wcat, bcat)
    q01, q23, k01, k23, v01, v23, skip = outs

    num2, asum2, _ = _attn_sc(src_g, dst_g, dst_s, q01, q23, k01, k23,
                              v01, v23)

    skip_p = jnp.pad(skip, ((0, NPAD - N), (0, 0)))
    fcw_p = jnp.pad(fc_w, ((0, 0), (0, 125)))
    fcb_p = jnp.pad(fc_b, (0, 125))[None]
    out = pl.pallas_call(
        _final_body,
        grid=(NPAD // 1024,),
        in_specs=[pl.BlockSpec((2, 1024, DH), lambda i: (0, i, 0)),
                  pl.BlockSpec((2, 2, 1024), lambda i: (0, 0, i)),
                  pl.BlockSpec((1024, 256), lambda i: (i, 0)),
                  pl.BlockSpec((1, 256), lambda i: (0, 0)),
                  pl.BlockSpec((256, 128), lambda i: (0, 0)),
                  pl.BlockSpec((1, 128), lambda i: (0, 0))],
        out_specs=pl.BlockSpec((1, 128), lambda i: (0, 0)),
        out_shape=jax.ShapeDtypeStruct((1, 128), jnp.float32),
        scratch_shapes=[pltpu.VMEM((1, 256), jnp.float32)],
    )(num2, asum2, skip_p, bskip[None], fcw_p, fcb_p)
    return out[:, :3]
